# Initial kernel scaffold; baseline (speedup 1.0000x reference)
import jax
import jax.numpy as jnp
from jax.experimental import pallas as pl


def kernel(x, edge_index, W1, b1, Wa, a_src, a_dst, ba, W3, b3):
    raise NotImplementedError("write your pallas kernel here")


# speedup vs baseline: 36.2718x; 36.2718x over previous
import functools

import jax
import jax.numpy as jnp
from jax import lax
from jax.experimental import pallas as pl
from jax.experimental.pallas import tpu as pltpu
from jax.experimental.pallas import tpu_sc as plsc

N = 10000
NP = 10008
E = 320000
ERP = 2560
EPAD = ERP * 128
RW = ERP // 32
RT = ERP // 16
IN_D = 128
HID = 64
HEADS = 4

_mesh = plsc.VectorSubcoreMesh(core_axis_name="c", subcore_axis_name="s")
_sc_params = pltpu.CompilerParams(use_tc_tiling_on_sc=False,
                                 needs_layout_passes=False)
f32 = jnp.float32


def _ids():
    c = lax.axis_index("c")
    s = lax.axis_index("s")
    return c, s


@functools.partial(
    pl.kernel,
    out_type=[jax.ShapeDtypeStruct((NP,), f32),
              jax.ShapeDtypeStruct((NP,), f32)],
    mesh=_mesh,
    compiler_params=_sc_params,
    scratch_types=[
        pltpu.VMEM((RW, 128), jnp.int32),
        pltpu.VMEM((128,), f32),
        pltpu.VMEM_SHARED((NP,), f32),
    ],
)
def _sc_hist(dst_hbm, zeros_hbm, out0_hbm, out1_hbm, idx_v, ones_v, acc_sp):
    c, s = _ids()
    wid = s * 2 + c

    @pl.when(s == 0)
    def _():
        pltpu.sync_copy(zeros_hbm, acc_sp)

    for i in range(8):
        ones_v[pl.ds(i * 16, 16)] = jnp.ones((16,), f32)
    pltpu.sync_copy(dst_hbm.at[pl.ds(wid * RW, RW)], idx_v)
    plsc.subcore_barrier()

    def body(t, carry):
        pltpu.sync_copy(ones_v, acc_sp.at[idx_v.at[t]], add=True)
        return carry

    lax.fori_loop(0, RW, body, 0)
    plsc.subcore_barrier()

    @pl.when((s == 0) & (c == 0))
    def _():
        pltpu.sync_copy(acc_sp, out0_hbm)

    @pl.when((s == 0) & (c == 1))
    def _():
        pltpu.sync_copy(acc_sp, out1_hbm)


@functools.partial(
    pl.kernel,
    out_type=jax.ShapeDtypeStruct((2, NP, HID), f32),
    mesh=_mesh,
    compiler_params=_sc_params,
    scratch_types=[
        pltpu.VMEM((RW, 128), jnp.int32),
        pltpu.VMEM((RW, 128), jnp.int32),
        pltpu.VMEM((128, HID), f32),
        pltpu.SemaphoreType.DMA,
        pltpu.VMEM_SHARED((NP, HID), f32),
    ],
)
def _sc_gcn1(src_hbm, dst_hbm, hs_hbm, zeros_hbm, out_hbm,
             idxs_v, idxd_v, rows_v, sem, acc_sp):
    c, s = _ids()
    wid = s * 2 + c

    @pl.when(s == 0)
    def _():
        pltpu.sync_copy(zeros_hbm, acc_sp)

    pltpu.sync_copy(src_hbm.at[pl.ds(wid * RW, RW)], idxs_v)
    pltpu.sync_copy(dst_hbm.at[pl.ds(wid * RW, RW)], idxd_v)
    plsc.subcore_barrier()

    def body(t, carry):
        pltpu.async_copy(hs_hbm.at[idxs_v.at[t]], rows_v, sem).wait()
        pltpu.sync_copy(rows_v, acc_sp.at[idxd_v.at[t]], add=True)
        return carry

    lax.fori_loop(0, RW, body, 0)
    plsc.subcore_barrier()

    @pl.when(s == 0)
    def _():
        pltpu.sync_copy(acc_sp, out_hbm.at[c])


@functools.partial(
    pl.kernel,
    out_type=[
        jax.ShapeDtypeStruct((HEADS, ERP, 128), f32),
        jax.ShapeDtypeStruct((2, 128, 16), f32),
    ],
    mesh=_mesh,
    compiler_params=_sc_params,
    scratch_types=[
        pltpu.VMEM((RT, 128), jnp.int32),
        pltpu.VMEM((RT, 128), jnp.int32),
        pltpu.VMEM((NP,), f32),
        pltpu.VMEM((NP,), f32),
        pltpu.VMEM((NP,), f32),
        pltpu.VMEM((NP,), f32),
        pltpu.VMEM((RT, 128), f32),
        pltpu.VMEM((RT, 128), f32),
        pltpu.VMEM((8, 16), f32),
    ],
)
def _sc_gat_logits(src_hbm, dst_hbm, als0_h, als1_h, als2_h, als3_h,
                   ald0_h, ald1_h, ald2_h, ald3_h, e_hbm, tmax_hbm,
                   idxs_v, idxd_v, ta0, ta1, td0, td1, eb0, eb1, mxv):
    c, s = _ids()

    @pl.when(c == 0)
    def _():
        pltpu.sync_copy(als0_h, ta0)
        pltpu.sync_copy(als1_h, ta1)
        pltpu.sync_copy(ald0_h, td0)
        pltpu.sync_copy(ald1_h, td1)

    @pl.when(c == 1)
    def _():
        pltpu.sync_copy(als2_h, ta0)
        pltpu.sync_copy(als3_h, ta1)
        pltpu.sync_copy(ald2_h, td0)
        pltpu.sync_copy(ald3_h, td1)

    pltpu.sync_copy(src_hbm.at[pl.ds(s * RT, RT)], idxs_v)
    pltpu.sync_copy(dst_hbm.at[pl.ds(s * RT, RT)], idxd_v)

    def row(t, mx):
        def sub(i, mx):
            sv = idxs_v[t, pl.ds(i * 16, 16)]
            dv = idxd_v[t, pl.ds(i * 16, 16)]
            e0 = plsc.load_gather(ta0, [sv]) + plsc.load_gather(td0, [dv])
            e0 = jnp.maximum(e0, 0.2 * e0)
            e1 = plsc.load_gather(ta1, [sv]) + plsc.load_gather(td1, [dv])
            e1 = jnp.maximum(e1, 0.2 * e1)
            eb0[t, pl.ds(i * 16, 16)] = e0
            eb1[t, pl.ds(i * 16, 16)] = e1
            return jnp.maximum(mx, jnp.maximum(e0, e1))

        return lax.fori_loop(0, 8, sub, mx)

    mx = lax.fori_loop(0, RT, row, jnp.full((16,), -3.0e38, f32))
    for r in range(8):
        mxv[r] = mx
    pltpu.sync_copy(eb0, e_hbm.at[2 * c, pl.ds(s * RT, RT)])
    pltpu.sync_copy(eb1, e_hbm.at[2 * c + 1, pl.ds(s * RT, RT)])
    pltpu.sync_copy(mxv, tmax_hbm.at[c, pl.ds(s * 8, 8)])


@functools.partial(
    pl.kernel,
    out_type=[
        jax.ShapeDtypeStruct((2, NP, 128), f32),
        jax.ShapeDtypeStruct((NP,), f32),
        jax.ShapeDtypeStruct((NP,), f32),
        jax.ShapeDtypeStruct((NP,), f32),
        jax.ShapeDtypeStruct((NP,), f32),
    ],
    mesh=_mesh,
    compiler_params=_sc_params,
    scratch_types=[
        pltpu.VMEM((8, 128), jnp.int32),
        pltpu.VMEM((8, 128), jnp.int32),
        pltpu.VMEM((8, 128), f32),
        pltpu.VMEM((8, 128), f32),
        pltpu.VMEM((16,), f32),
        pltpu.VMEM((128, 128), f32),
        pltpu.SemaphoreType.DMA,
        pltpu.VMEM_SHARED((NP, 128), f32),
        pltpu.VMEM_SHARED((NP,), f32),
        pltpu.VMEM_SHARED((NP,), f32),
    ],
)
def _sc_gat_main(src2_hbm, dst_hbm, e_hbm, m_hbm, h2_hbm,
                 zeros2_hbm, zeros1_hbm,
                 outg_hbm, outs0_h, outs1_h, outs2_h, outs3_h,
                 idxs_v, idxd_v, e0_v, e1_v, m_v, rows_v, sem,
                 acc_sp, s0_sp, s1_sp):
    c, s = _ids()

    @pl.when(s == 0)
    def _():
        pltpu.sync_copy(zeros2_hbm, acc_sp)
        pltpu.sync_copy(zeros1_hbm, s0_sp)
        pltpu.sync_copy(zeros1_hbm, s1_sp)

    pltpu.sync_copy(m_hbm, m_v)
    plsc.subcore_barrier()
    mv = m_v[...]
    base = s * RT

    def chunk(t, carry):
        row0 = base + t * 8
        pltpu.sync_copy(src2_hbm.at[c, pl.ds(row0, 8)], idxs_v)
        pltpu.sync_copy(dst_hbm.at[pl.ds(row0, 8)], idxd_v)
        pltpu.sync_copy(e_hbm.at[2 * c, pl.ds(row0, 8)], e0_v)
        pltpu.sync_copy(e_hbm.at[2 * c + 1, pl.ds(row0, 8)], e1_v)

        def row(r, cc):
            pltpu.async_copy(h2_hbm.at[idxs_v.at[r]], rows_v, sem).wait()

            def expi(i, c2):
                e0_v[r, pl.ds(i * 16, 16)] = jnp.exp(e0_v[r, pl.ds(i * 16, 16)] - mv)
                e1_v[r, pl.ds(i * 16, 16)] = jnp.exp(e1_v[r, pl.ds(i * 16, 16)] - mv)
                return c2

            lax.fori_loop(0, 8, expi, 0)
            r16 = jnp.full((16,), r, jnp.int32)

            def scale(i, c2):
                i16 = jnp.full((16,), i, jnp.int32)
                b0 = plsc.load_gather(e0_v, [r16, i16])
                b1 = plsc.load_gather(e1_v, [r16, i16])
                for j in range(4):
                    rows_v[i, pl.ds(j * 16, 16)] = rows_v[i, pl.ds(j * 16, 16)] * b0
                for j in range(4, 8):
                    rows_v[i, pl.ds(j * 16, 16)] = rows_v[i, pl.ds(j * 16, 16)] * b1
                return c2

            lax.fori_loop(0, 128, scale, 0)
            pltpu.sync_copy(rows_v, acc_sp.at[idxd_v.at[r]], add=True)
            pltpu.sync_copy(e0_v.at[r], s0_sp.at[idxd_v.at[r]], add=True)
            pltpu.sync_copy(e1_v.at[r], s1_sp.at[idxd_v.at[r]], add=True)
            return cc

        lax.fori_loop(0, 8, row, 0)
        return carry

    lax.fori_loop(0, RT // 8, chunk, 0)
    plsc.subcore_barrier()

    @pl.when(s == 0)
    def _():
        pltpu.sync_copy(acc_sp, outg_hbm.at[c])

    @pl.when((s == 0) & (c == 0))
    def _():
        pltpu.sync_copy(s0_sp, outs0_h)
        pltpu.sync_copy(s1_sp, outs1_h)

    @pl.when((s == 0) & (c == 1))
    def _():
        pltpu.sync_copy(s0_sp, outs2_h)
        pltpu.sync_copy(s1_sp, outs3_h)


@functools.partial(
    pl.kernel,
    out_type=[jax.ShapeDtypeStruct((NP,), f32),
              jax.ShapeDtypeStruct((NP,), f32)],
    mesh=_mesh,
    compiler_params=_sc_params,
    scratch_types=[
        pltpu.VMEM((RW, 128), jnp.int32),
        pltpu.VMEM((RW, 128), jnp.int32),
        pltpu.VMEM((NP,), f32),
        pltpu.VMEM((128,), f32),
        pltpu.VMEM_SHARED((NP,), f32),
    ],
)
def _sc_gcn2(src_hbm, dst_hbm, zs_hbm, zeros_hbm, out0_hbm, out1_hbm,
             idxs_v, idxd_v, zs_v, upd_v, acc_sp):
    c, s = _ids()
    wid = s * 2 + c

    @pl.when(s == 0)
    def _():
        pltpu.sync_copy(zeros_hbm, acc_sp)

    pltpu.sync_copy(zs_hbm, zs_v)
    pltpu.sync_copy(src_hbm.at[pl.ds(wid * RW, RW)], idxs_v)
    pltpu.sync_copy(dst_hbm.at[pl.ds(wid * RW, RW)], idxd_v)
    plsc.subcore_barrier()

    def body(t, carry):
        def sub(i, cc):
            sv = idxs_v[t, pl.ds(i * 16, 16)]
            upd_v[pl.ds(i * 16, 16)] = plsc.load_gather(zs_v, [sv])
            return cc

        lax.fori_loop(0, 8, sub, 0)
        pltpu.sync_copy(upd_v, acc_sp.at[idxd_v.at[t]], add=True)
        return carry

    lax.fori_loop(0, RW, body, 0)
    plsc.subcore_barrier()

    @pl.when((s == 0) & (c == 0))
    def _():
        pltpu.sync_copy(acc_sp, out0_hbm)

    @pl.when((s == 0) & (c == 1))
    def _():
        pltpu.sync_copy(acc_sp, out1_hbm)


def _tc1_body(x_ref, w_ref, o_ref):
    o_ref[...] = jnp.dot(x_ref[...], w_ref[...], preferred_element_type=f32)


def _tc2_body(dp0_ref, dp1_ref, h1_ref, dinv_ref, hs_ref):
    deg = dp0_ref[...] + dp1_ref[...] + 1.0
    dinv = lax.rsqrt(deg)
    dinv_ref[...] = dinv
    hs_ref[...] = h1_ref[...] * dinv[:, None]


def _tc3_body(agg_ref, hs_ref, dinv_ref, b1_ref, wa_ref, asrc_ref, adst_ref,
              h2_ref, als_ref, ald_ref, eself_ref):
    g1 = jnp.maximum(
        dinv_ref[...][:, None] * (agg_ref[0] + agg_ref[1] + hs_ref[...])
        + b1_ref[...][None, :], 0.0)
    wa = wa_ref[...]
    h2_ref[0] = jnp.dot(g1, wa[:, :128], preferred_element_type=f32)
    h2_ref[1] = jnp.dot(g1, wa[:, 128:], preferred_element_type=f32)
    va = jnp.stack([wa[:, 64 * k:64 * k + 64] @ asrc_ref[k]
                    for k in range(HEADS)], axis=1)
    vd = jnp.stack([wa[:, 64 * k:64 * k + 64] @ adst_ref[k]
                    for k in range(HEADS)], axis=1)
    als = jnp.dot(g1, va, preferred_element_type=f32).T
    ald = jnp.dot(g1, vd, preferred_element_type=f32).T
    als_ref[...] = als
    ald_ref[...] = ald
    es = als + ald
    eself_ref[...] = jnp.maximum(es, 0.2 * es)


def _tc4_body(tmax_ref, eself_ref, m_ref):
    m = jnp.maximum(jnp.max(tmax_ref[...]), jnp.max(eself_ref[...]))
    m_ref[...] = jnp.full((16,), m, f32)


def _tc5_body(outg_ref, outs_ref, eself_ref, m_ref, h2_ref, dinv_ref,
              ba_ref, w3_ref, zs_ref):
    m = m_ref[...][0]
    exs = jnp.exp(eself_ref[...] - m)
    o2 = jnp.zeros((NP, HID), f32)
    for c in range(2):
        for j in range(2):
            k = 2 * c + j
            hk = h2_ref[c][:, 64 * j:64 * j + 64]
            rawk = outg_ref[c][:, 64 * j:64 * j + 64] + exs[k][:, None] * hk
            sk = outs_ref[k] + exs[k]
            o2 = o2 + rawk / (sk[:, None] + 1e-16)
    g2 = jnp.maximum(0.25 * o2 + ba_ref[...][None, :], 0.0)
    z = jnp.dot(g2, w3_ref[...], preferred_element_type=f32)
    zs_ref[...] = z[:, 0] * dinv_ref[...]


def _tc6_body(az0_ref, az1_ref, zs_ref, dinv_ref, b3_ref, o_ref):
    val = dinv_ref[...] * (az0_ref[...] + az1_ref[...] + zs_ref[...]) + b3_ref[0]
    o_ref[...] = (1.0 / (1.0 + jnp.exp(-val)))[:, None]


@jax.jit
def _run(x, edge_index, W1, b1, Wa, a_src, a_dst, ba, W3, b3):
    src = edge_index[0].astype(jnp.int32)
    dst = edge_index[1].astype(jnp.int32)
    pad = jnp.full((EPAD - E,), N, jnp.int32)
    src2d = jnp.concatenate([src, pad]).reshape(ERP, 128)
    dst2d = jnp.concatenate([dst, pad]).reshape(ERP, 128)
    srcs2 = jnp.stack([src2d, src2d + NP])

    xp = jnp.pad(x, ((0, NP - N), (0, 0)))
    zeros1 = jnp.zeros((NP,), f32)
    zeros64 = jnp.zeros((NP, HID), f32)
    zeros128 = jnp.zeros((NP, 128), f32)

    h1 = pl.pallas_call(
        _tc1_body,
        out_shape=jax.ShapeDtypeStruct((NP, HID), f32),
    )(xp, W1)

    dp0, dp1 = _sc_hist(dst2d, zeros1)

    dinv, hs = pl.pallas_call(
        _tc2_body,
        out_shape=[jax.ShapeDtypeStruct((NP,), f32),
                   jax.ShapeDtypeStruct((NP, HID), f32)],
    )(dp0, dp1, h1)

    agg = _sc_gcn1(src2d, dst2d, hs, zeros64)

    h2r, als, ald, eself = pl.pallas_call(
        _tc3_body,
        out_shape=[jax.ShapeDtypeStruct((2, NP, 128), f32),
                   jax.ShapeDtypeStruct((HEADS, NP), f32),
                   jax.ShapeDtypeStruct((HEADS, NP), f32),
                   jax.ShapeDtypeStruct((HEADS, NP), f32)],
    )(agg, hs, dinv, b1, Wa, a_src, a_dst)

    e_edges, tmax = _sc_gat_logits(
        src2d, dst2d, als[0], als[1], als[2], als[3],
        ald[0], ald[1], ald[2], ald[3])

    m16 = pl.pallas_call(
        _tc4_body,
        out_shape=jax.ShapeDtypeStruct((16,), f32),
    )(tmax, eself)

    h2flat = h2r.reshape(2 * NP, 128)
    outg, s0, s1, s2, s3 = _sc_gat_main(srcs2, dst2d, e_edges, m16, h2flat,
                                        zeros128, zeros1)
    outs = jnp.stack([s0, s1, s2, s3])

    zs = pl.pallas_call(
        _tc5_body,
        out_shape=jax.ShapeDtypeStruct((NP,), f32),
    )(outg, outs, eself, m16, h2r, dinv, ba, W3)

    az0, az1 = _sc_gcn2(src2d, dst2d, zs, zeros1)

    out = pl.pallas_call(
        _tc6_body,
        out_shape=jax.ShapeDtypeStruct((NP, 1), f32),
    )(az0, az1, zs, dinv, b3)

    return out[:N]


def kernel(x, edge_index, W1, b1, Wa, a_src, a_dst, ba, W3, b3):
    return _run(x, edge_index, W1, b1, Wa, a_src, a_dst, ba, W3, b3)


# speedup vs baseline: 47.9744x; 1.3226x over previous
import functools

import jax
import jax.numpy as jnp
from jax import lax
from jax.experimental import pallas as pl
from jax.experimental.pallas import tpu as pltpu
from jax.experimental.pallas import tpu_sc as plsc

N = 10000
NP = 10008
E = 320000
ERP = 2560
EPAD = ERP * 128
RW = ERP // 32
RT = ERP // 16
IN_D = 128
HID = 64
HEADS = 4

_mesh = plsc.VectorSubcoreMesh(core_axis_name="c", subcore_axis_name="s")
_sc_params = pltpu.CompilerParams(use_tc_tiling_on_sc=False,
                                 needs_layout_passes=False)
f32 = jnp.float32


def _ids():
    c = lax.axis_index("c")
    s = lax.axis_index("s")
    return c, s


@functools.partial(
    pl.kernel,
    out_type=[jax.ShapeDtypeStruct((NP,), f32),
              jax.ShapeDtypeStruct((NP,), f32)],
    mesh=_mesh,
    compiler_params=_sc_params,
    scratch_types=[
        pltpu.VMEM((RW, 128), jnp.int32),
        pltpu.VMEM((128,), f32),
        pltpu.VMEM_SHARED((NP,), f32),
    ],
)
def _sc_hist(dst_hbm, zeros_hbm, out0_hbm, out1_hbm, idx_v, ones_v, acc_sp):
    c, s = _ids()
    wid = s * 2 + c

    @pl.when(s == 0)
    def _():
        pltpu.sync_copy(zeros_hbm, acc_sp)

    for i in range(8):
        ones_v[pl.ds(i * 16, 16)] = jnp.ones((16,), f32)
    pltpu.sync_copy(dst_hbm.at[pl.ds(wid * RW, RW)], idx_v)
    plsc.subcore_barrier()

    def body(t, carry):
        pltpu.sync_copy(ones_v, acc_sp.at[idx_v.at[t]], add=True)
        return carry

    lax.fori_loop(0, RW, body, 0)
    plsc.subcore_barrier()

    @pl.when((s == 0) & (c == 0))
    def _():
        pltpu.sync_copy(acc_sp, out0_hbm)

    @pl.when((s == 0) & (c == 1))
    def _():
        pltpu.sync_copy(acc_sp, out1_hbm)


@functools.partial(
    pl.kernel,
    out_type=jax.ShapeDtypeStruct((2, NP, HID), f32),
    mesh=_mesh,
    compiler_params=_sc_params,
    scratch_types=[
        pltpu.VMEM((RW, 128), jnp.int32),
        pltpu.VMEM((RW, 128), jnp.int32),
        pltpu.VMEM((128, HID), f32),
        pltpu.VMEM((128, HID), f32),
        pltpu.SemaphoreType.DMA,
        pltpu.SemaphoreType.DMA,
        pltpu.VMEM_SHARED((NP, HID), f32),
    ],
)
def _sc_gcn1(src_hbm, dst_hbm, hs_hbm, zeros_hbm, out_hbm,
             idxs_v, idxd_v, rows0_v, rows1_v, sem0, sem1, acc_sp):
    c, s = _ids()
    wid = s * 2 + c

    @pl.when(s == 0)
    def _():
        pltpu.sync_copy(zeros_hbm, acc_sp)

    pltpu.sync_copy(src_hbm.at[pl.ds(wid * RW, RW)], idxs_v)
    pltpu.sync_copy(dst_hbm.at[pl.ds(wid * RW, RW)], idxd_v)
    plsc.subcore_barrier()

    rows = (rows0_v, rows1_v)
    sems = (sem0, sem1)
    pltpu.async_copy(hs_hbm.at[idxs_v.at[0]], rows0_v, sem0)

    def body(ko, carry):
        for b in range(2):
            t = 2 * ko + b
            nb = 1 - b

            @pl.when(t + 1 < RW)
            def _():
                pltpu.async_copy(hs_hbm.at[idxs_v.at[t + 1]], rows[nb], sems[nb])

            pltpu.make_async_copy(hs_hbm.at[idxs_v.at[t]], rows[b], sems[b]).wait()
            pltpu.sync_copy(rows[b], acc_sp.at[idxd_v.at[t]], add=True)
        return carry

    lax.fori_loop(0, RW // 2, body, 0)
    plsc.subcore_barrier()

    @pl.when(s == 0)
    def _():
        pltpu.sync_copy(acc_sp, out_hbm.at[c])


@functools.partial(
    pl.kernel,
    out_type=[
        jax.ShapeDtypeStruct((HEADS, ERP, 128), f32),
        jax.ShapeDtypeStruct((2, 128, 16), f32),
    ],
    mesh=_mesh,
    compiler_params=_sc_params,
    scratch_types=[
        pltpu.VMEM((RT, 128), jnp.int32),
        pltpu.VMEM((RT, 128), jnp.int32),
        pltpu.VMEM((NP,), f32),
        pltpu.VMEM((NP,), f32),
        pltpu.VMEM((NP,), f32),
        pltpu.VMEM((NP,), f32),
        pltpu.VMEM((RT, 128), f32),
        pltpu.VMEM((RT, 128), f32),
        pltpu.VMEM((8, 16), f32),
    ],
)
def _sc_gat_logits(src_hbm, dst_hbm, als0_h, als1_h, als2_h, als3_h,
                   ald0_h, ald1_h, ald2_h, ald3_h, e_hbm, tmax_hbm,
                   idxs_v, idxd_v, ta0, ta1, td0, td1, eb0, eb1, mxv):
    c, s = _ids()

    @pl.when(c == 0)
    def _():
        pltpu.sync_copy(als0_h, ta0)
        pltpu.sync_copy(als1_h, ta1)
        pltpu.sync_copy(ald0_h, td0)
        pltpu.sync_copy(ald1_h, td1)

    @pl.when(c == 1)
    def _():
        pltpu.sync_copy(als2_h, ta0)
        pltpu.sync_copy(als3_h, ta1)
        pltpu.sync_copy(ald2_h, td0)
        pltpu.sync_copy(ald3_h, td1)

    pltpu.sync_copy(src_hbm.at[pl.ds(s * RT, RT)], idxs_v)
    pltpu.sync_copy(dst_hbm.at[pl.ds(s * RT, RT)], idxd_v)

    def row(t, mx):
        def sub(i, mx):
            sv = idxs_v[t, pl.ds(i * 16, 16)]
            dv = idxd_v[t, pl.ds(i * 16, 16)]
            e0 = plsc.load_gather(ta0, [sv]) + plsc.load_gather(td0, [dv])
            e0 = jnp.maximum(e0, 0.2 * e0)
            e1 = plsc.load_gather(ta1, [sv]) + plsc.load_gather(td1, [dv])
            e1 = jnp.maximum(e1, 0.2 * e1)
            eb0[t, pl.ds(i * 16, 16)] = e0
            eb1[t, pl.ds(i * 16, 16)] = e1
            return jnp.maximum(mx, jnp.maximum(e0, e1))

        return lax.fori_loop(0, 8, sub, mx)

    mx = lax.fori_loop(0, RT, row, jnp.full((16,), -3.0e38, f32))
    for r in range(8):
        mxv[r] = mx
    pltpu.sync_copy(eb0, e_hbm.at[2 * c, pl.ds(s * RT, RT)])
    pltpu.sync_copy(eb1, e_hbm.at[2 * c + 1, pl.ds(s * RT, RT)])
    pltpu.sync_copy(mxv, tmax_hbm.at[c, pl.ds(s * 8, 8)])


@functools.partial(
    pl.kernel,
    out_type=[
        jax.ShapeDtypeStruct((2, NP, 128), f32),
        jax.ShapeDtypeStruct((NP,), f32),
        jax.ShapeDtypeStruct((NP,), f32),
        jax.ShapeDtypeStruct((NP,), f32),
        jax.ShapeDtypeStruct((NP,), f32),
    ],
    mesh=_mesh,
    compiler_params=_sc_params,
    scratch_types=[
        pltpu.VMEM((16, 128), jnp.int32),
        pltpu.VMEM((16, 128), jnp.int32),
        pltpu.VMEM((16, 128), f32),
        pltpu.VMEM((16, 128), f32),
        pltpu.VMEM((16,), f32),
        pltpu.VMEM((128, 128), f32),
        pltpu.VMEM((128, 128), f32),
        pltpu.SemaphoreType.DMA,
        pltpu.SemaphoreType.DMA,
        pltpu.VMEM_SHARED((NP, 128), f32),
        pltpu.VMEM_SHARED((NP,), f32),
        pltpu.VMEM_SHARED((NP,), f32),
    ],
)
def _sc_gat_main(src2_hbm, dst_hbm, e_hbm, m_hbm, h2_hbm,
                 zeros2_hbm, zeros1_hbm,
                 outg_hbm, outs0_h, outs1_h, outs2_h, outs3_h,
                 idxs_v, idxd_v, e0_v, e1_v, m_v, rows0_v, rows1_v,
                 sem0, sem1, acc_sp, s0_sp, s1_sp):
    c, s = _ids()

    @pl.when(s == 0)
    def _():
        pltpu.sync_copy(zeros2_hbm, acc_sp)
        pltpu.sync_copy(zeros1_hbm, s0_sp)
        pltpu.sync_copy(zeros1_hbm, s1_sp)

    pltpu.sync_copy(m_hbm, m_v)
    plsc.subcore_barrier()
    mv = m_v[...]
    base = s * RT
    rows = (rows0_v, rows1_v)
    sems = (sem0, sem1)

    def stage(g, sl0):
        row0 = base + g * 8
        pltpu.sync_copy(src2_hbm.at[c, pl.ds(row0, 8)], idxs_v.at[pl.ds(sl0, 8)])
        pltpu.sync_copy(dst_hbm.at[pl.ds(row0, 8)], idxd_v.at[pl.ds(sl0, 8)])
        pltpu.sync_copy(e_hbm.at[2 * c, pl.ds(row0, 8)], e0_v.at[pl.ds(sl0, 8)])
        pltpu.sync_copy(e_hbm.at[2 * c + 1, pl.ds(row0, 8)], e1_v.at[pl.ds(sl0, 8)])

    stage(0, 0)
    pltpu.async_copy(h2_hbm.at[idxs_v.at[0]], rows0_v, sem0)

    def body(ko, carry):
        for b in range(2):
            t = 2 * ko + b
            nb = 1 - b
            g = t // 8
            r = t - g * 8
            par = g % 2
            sl = par * 8 + r

            @pl.when((r == 7) & (t + 1 < RT))
            def _():
                stage(g + 1, (1 - par) * 8)

            tn = t + 1
            gn = tn // 8
            sln = (gn % 2) * 8 + (tn - gn * 8)

            @pl.when(t + 1 < RT)
            def _():
                pltpu.async_copy(h2_hbm.at[idxs_v.at[sln]], rows[nb], sems[nb])

            pltpu.make_async_copy(h2_hbm.at[idxs_v.at[sl]], rows[b], sems[b]).wait()

            def expi(i, c2):
                e0_v[sl, pl.ds(i * 16, 16)] = jnp.exp(e0_v[sl, pl.ds(i * 16, 16)] - mv)
                e1_v[sl, pl.ds(i * 16, 16)] = jnp.exp(e1_v[sl, pl.ds(i * 16, 16)] - mv)
                return c2

            lax.fori_loop(0, 8, expi, 0)
            sl16 = jnp.full((16,), sl, jnp.int32)
            rv = rows[b]

            def scale(i, c2):
                for u in range(2):
                    iu = 2 * i + u
                    i16 = jnp.full((16,), iu, jnp.int32)
                    b0 = plsc.load_gather(e0_v, [sl16, i16])
                    b1 = plsc.load_gather(e1_v, [sl16, i16])
                    for j in range(4):
                        rv[iu, pl.ds(j * 16, 16)] = rv[iu, pl.ds(j * 16, 16)] * b0
                    for j in range(4, 8):
                        rv[iu, pl.ds(j * 16, 16)] = rv[iu, pl.ds(j * 16, 16)] * b1
                return c2

            lax.fori_loop(0, 64, scale, 0)

            pltpu.sync_copy(e0_v.at[sl], s0_sp.at[idxd_v.at[sl]], add=True)
            pltpu.sync_copy(e1_v.at[sl], s1_sp.at[idxd_v.at[sl]], add=True)
            pltpu.sync_copy(rv, acc_sp.at[idxd_v.at[sl]], add=True)
        return carry

    lax.fori_loop(0, RT // 2, body, 0)
    plsc.subcore_barrier()

    @pl.when(s == 0)
    def _():
        pltpu.sync_copy(acc_sp, outg_hbm.at[c])

    @pl.when((s == 0) & (c == 0))
    def _():
        pltpu.sync_copy(s0_sp, outs0_h)
        pltpu.sync_copy(s1_sp, outs1_h)

    @pl.when((s == 0) & (c == 1))
    def _():
        pltpu.sync_copy(s0_sp, outs2_h)
        pltpu.sync_copy(s1_sp, outs3_h)


@functools.partial(
    pl.kernel,
    out_type=[jax.ShapeDtypeStruct((NP,), f32),
              jax.ShapeDtypeStruct((NP,), f32)],
    mesh=_mesh,
    compiler_params=_sc_params,
    scratch_types=[
        pltpu.VMEM((RW, 128), jnp.int32),
        pltpu.VMEM((RW, 128), jnp.int32),
        pltpu.VMEM((NP,), f32),
        pltpu.VMEM((128,), f32),
        pltpu.VMEM_SHARED((NP,), f32),
    ],
)
def _sc_gcn2(src_hbm, dst_hbm, zs_hbm, zeros_hbm, out0_hbm, out1_hbm,
             idxs_v, idxd_v, zs_v, upd_v, acc_sp):
    c, s = _ids()
    wid = s * 2 + c

    @pl.when(s == 0)
    def _():
        pltpu.sync_copy(zeros_hbm, acc_sp)

    pltpu.sync_copy(zs_hbm, zs_v)
    pltpu.sync_copy(src_hbm.at[pl.ds(wid * RW, RW)], idxs_v)
    pltpu.sync_copy(dst_hbm.at[pl.ds(wid * RW, RW)], idxd_v)
    plsc.subcore_barrier()

    def body(t, carry):
        def sub(i, cc):
            sv = idxs_v[t, pl.ds(i * 16, 16)]
            upd_v[pl.ds(i * 16, 16)] = plsc.load_gather(zs_v, [sv])
            return cc

        lax.fori_loop(0, 8, sub, 0)
        pltpu.sync_copy(upd_v, acc_sp.at[idxd_v.at[t]], add=True)
        return carry

    lax.fori_loop(0, RW, body, 0)
    plsc.subcore_barrier()

    @pl.when((s == 0) & (c == 0))
    def _():
        pltpu.sync_copy(acc_sp, out0_hbm)

    @pl.when((s == 0) & (c == 1))
    def _():
        pltpu.sync_copy(acc_sp, out1_hbm)


def _tc1_body(x_ref, w_ref, o_ref):
    o_ref[...] = jnp.dot(x_ref[...], w_ref[...], preferred_element_type=f32)


def _tc2_body(dp0_ref, dp1_ref, h1_ref, dinv_ref, hs_ref):
    deg = dp0_ref[...] + dp1_ref[...] + 1.0
    dinv = lax.rsqrt(deg)
    dinv_ref[...] = dinv
    hs_ref[...] = h1_ref[...] * dinv[:, None]


def _tc3_body(agg_ref, hs_ref, dinv_ref, b1_ref, wa_ref, asrc_ref, adst_ref,
              h2_ref, als_ref, ald_ref, eself_ref):
    g1 = jnp.maximum(
        dinv_ref[...][:, None] * (agg_ref[0] + agg_ref[1] + hs_ref[...])
        + b1_ref[...][None, :], 0.0)
    wa = wa_ref[...]
    h2_ref[0] = jnp.dot(g1, wa[:, :128], preferred_element_type=f32)
    h2_ref[1] = jnp.dot(g1, wa[:, 128:], preferred_element_type=f32)
    va = jnp.stack([wa[:, 64 * k:64 * k + 64] @ asrc_ref[k]
                    for k in range(HEADS)], axis=1)
    vd = jnp.stack([wa[:, 64 * k:64 * k + 64] @ adst_ref[k]
                    for k in range(HEADS)], axis=1)
    als = jnp.dot(g1, va, preferred_element_type=f32).T
    ald = jnp.dot(g1, vd, preferred_element_type=f32).T
    als_ref[...] = als
    ald_ref[...] = ald
    es = als + ald
    eself_ref[...] = jnp.maximum(es, 0.2 * es)


def _tc4_body(tmax_ref, eself_ref, m_ref):
    m = jnp.maximum(jnp.max(tmax_ref[...]), jnp.max(eself_ref[...]))
    m_ref[...] = jnp.full((16,), m, f32)


def _tc5_body(outg_ref, outs_ref, eself_ref, m_ref, h2_ref, dinv_ref,
              ba_ref, w3_ref, zs_ref):
    m = m_ref[...][0]
    exs = jnp.exp(eself_ref[...] - m)
    o2 = jnp.zeros((NP, HID), f32)
    for c in range(2):
        for j in range(2):
            k = 2 * c + j
            hk = h2_ref[c][:, 64 * j:64 * j + 64]
            rawk = outg_ref[c][:, 64 * j:64 * j + 64] + exs[k][:, None] * hk
            sk = outs_ref[k] + exs[k]
            o2 = o2 + rawk / (sk[:, None] + 1e-16)
    g2 = jnp.maximum(0.25 * o2 + ba_ref[...][None, :], 0.0)
    z = jnp.dot(g2, w3_ref[...], preferred_element_type=f32)
    zs_ref[...] = z[:, 0] * dinv_ref[...]


def _tc6_body(az0_ref, az1_ref, zs_ref, dinv_ref, b3_ref, o_ref):
    val = dinv_ref[...] * (az0_ref[...] + az1_ref[...] + zs_ref[...]) + b3_ref[0]
    o_ref[...] = (1.0 / (1.0 + jnp.exp(-val)))[:, None]


@jax.jit
def _run(x, edge_index, W1, b1, Wa, a_src, a_dst, ba, W3, b3):
    src = edge_index[0].astype(jnp.int32)
    dst = edge_index[1].astype(jnp.int32)
    pad = jnp.full((EPAD - E,), N, jnp.int32)
    src2d = jnp.concatenate([src, pad]).reshape(ERP, 128)
    dst2d = jnp.concatenate([dst, pad]).reshape(ERP, 128)
    srcs2 = jnp.stack([src2d, src2d + NP])

    xp = jnp.pad(x, ((0, NP - N), (0, 0)))
    zeros1 = jnp.zeros((NP,), f32)
    zeros64 = jnp.zeros((NP, HID), f32)
    zeros128 = jnp.zeros((NP, 128), f32)

    h1 = pl.pallas_call(
        _tc1_body,
        out_shape=jax.ShapeDtypeStruct((NP, HID), f32),
    )(xp, W1)

    dp0, dp1 = _sc_hist(dst2d, zeros1)

    dinv, hs = pl.pallas_call(
        _tc2_body,
        out_shape=[jax.ShapeDtypeStruct((NP,), f32),
                   jax.ShapeDtypeStruct((NP, HID), f32)],
    )(dp0, dp1, h1)

    agg = _sc_gcn1(src2d, dst2d, hs, zeros64)

    h2r, als, ald, eself = pl.pallas_call(
        _tc3_body,
        out_shape=[jax.ShapeDtypeStruct((2, NP, 128), f32),
                   jax.ShapeDtypeStruct((HEADS, NP), f32),
                   jax.ShapeDtypeStruct((HEADS, NP), f32),
                   jax.ShapeDtypeStruct((HEADS, NP), f32)],
    )(agg, hs, dinv, b1, Wa, a_src, a_dst)

    e_edges, tmax = _sc_gat_logits(
        src2d, dst2d, als[0], als[1], als[2], als[3],
        ald[0], ald[1], ald[2], ald[3])

    m16 = pl.pallas_call(
        _tc4_body,
        out_shape=jax.ShapeDtypeStruct((16,), f32),
    )(tmax, eself)

    h2flat = h2r.reshape(2 * NP, 128)
    outg, s0, s1, s2, s3 = _sc_gat_main(srcs2, dst2d, e_edges, m16, h2flat,
                                        zeros128, zeros1)
    outs = jnp.stack([s0, s1, s2, s3])

    zs = pl.pallas_call(
        _tc5_body,
        out_shape=jax.ShapeDtypeStruct((NP,), f32),
    )(outg, outs, eself, m16, h2r, dinv, ba, W3)

    az0, az1 = _sc_gcn2(src2d, dst2d, zs, zeros1)

    out = pl.pallas_call(
        _tc6_body,
        out_shape=jax.ShapeDtypeStruct((NP, 1), f32),
    )(az0, az1, zs, dinv, b3)

    return out[:N]


def kernel(x, edge_index, W1, b1, Wa, a_src, a_dst, ba, W3, b3):
    return _run(x, edge_index, W1, b1, Wa, a_src, a_dst, ba, W3, b3)


# speedup vs baseline: 48.9313x; 1.0199x over previous
import functools

import jax
import jax.numpy as jnp
from jax import lax
from jax.experimental import pallas as pl
from jax.experimental.pallas import tpu as pltpu
from jax.experimental.pallas import tpu_sc as plsc

N = 10000
NP = 10008
E = 320000
ERP = 2560
EPAD = ERP * 128
RW = ERP // 32
RT = ERP // 16
IN_D = 128
HID = 64
HEADS = 4

_mesh = plsc.VectorSubcoreMesh(core_axis_name="c", subcore_axis_name="s")
_sc_params = pltpu.CompilerParams(use_tc_tiling_on_sc=False,
                                 needs_layout_passes=False)
f32 = jnp.float32


def _ids():
    c = lax.axis_index("c")
    s = lax.axis_index("s")
    return c, s


@functools.partial(
    pl.kernel,
    out_type=[jax.ShapeDtypeStruct((NP,), f32),
              jax.ShapeDtypeStruct((NP,), f32)],
    mesh=_mesh,
    compiler_params=_sc_params,
    scratch_types=[
        pltpu.VMEM((RW, 128), jnp.int32),
        pltpu.VMEM((128,), f32),
        pltpu.VMEM_SHARED((NP,), f32),
    ],
)
def _sc_hist(dst_hbm, zeros_hbm, out0_hbm, out1_hbm, idx_v, ones_v, acc_sp):
    c, s = _ids()
    wid = s * 2 + c

    @pl.when(s == 0)
    def _():
        pltpu.sync_copy(zeros_hbm, acc_sp)

    for i in range(8):
        ones_v[pl.ds(i * 16, 16)] = jnp.ones((16,), f32)
    pltpu.sync_copy(dst_hbm.at[pl.ds(wid * RW, RW)], idx_v)
    plsc.subcore_barrier()

    def body(t, carry):
        pltpu.sync_copy(ones_v, acc_sp.at[idx_v.at[t]], add=True)
        return carry

    lax.fori_loop(0, RW, body, 0)
    plsc.subcore_barrier()

    @pl.when((s == 0) & (c == 0))
    def _():
        pltpu.sync_copy(acc_sp, out0_hbm)

    @pl.when((s == 0) & (c == 1))
    def _():
        pltpu.sync_copy(acc_sp, out1_hbm)


@functools.partial(
    pl.kernel,
    out_type=jax.ShapeDtypeStruct((2, NP, HID), f32),
    mesh=_mesh,
    compiler_params=_sc_params,
    scratch_types=[
        pltpu.VMEM((RW, 128), jnp.int32),
        pltpu.VMEM((RW, 128), jnp.int32),
        pltpu.VMEM((128, HID), f32),
        pltpu.VMEM((128, HID), f32),
        pltpu.SemaphoreType.DMA,
        pltpu.SemaphoreType.DMA,
        pltpu.SemaphoreType.DMA,
        pltpu.SemaphoreType.DMA,
        pltpu.VMEM_SHARED((NP, HID), f32),
    ],
)
def _sc_gcn1(src_hbm, dst_hbm, hs_hbm, zeros_hbm, out_hbm,
             idxs_v, idxd_v, rows0_v, rows1_v, sem0, sem1, ssem0, ssem1,
             acc_sp):
    c, s = _ids()
    wid = s * 2 + c

    @pl.when(s == 0)
    def _():
        pltpu.sync_copy(zeros_hbm, acc_sp)

    pltpu.sync_copy(src_hbm.at[pl.ds(wid * RW, RW)], idxs_v)
    pltpu.sync_copy(dst_hbm.at[pl.ds(wid * RW, RW)], idxd_v)
    plsc.subcore_barrier()

    rows = (rows0_v, rows1_v)
    sems = (sem0, sem1)
    ssems = (ssem0, ssem1)
    pltpu.async_copy(hs_hbm.at[idxs_v.at[0]], rows0_v, sem0)

    def body(ko, carry):
        for b in range(2):
            t = 2 * ko + b
            nb = 1 - b

            @pl.when(t + 1 < RW)
            def _():
                @pl.when(t >= 1)
                def _():
                    pltpu.make_async_copy(
                        rows[nb], acc_sp.at[idxd_v.at[t]], ssems[nb]).wait()

                pltpu.async_copy(hs_hbm.at[idxs_v.at[t + 1]], rows[nb], sems[nb])

            pltpu.make_async_copy(hs_hbm.at[idxs_v.at[t]], rows[b], sems[b]).wait()
            pltpu.async_copy(rows[b], acc_sp.at[idxd_v.at[t]], ssems[b], add=True)
        return carry

    lax.fori_loop(0, RW // 2, body, 0)
    pltpu.make_async_copy(rows[0], acc_sp.at[idxd_v.at[0]], ssems[0]).wait()
    pltpu.make_async_copy(rows[1], acc_sp.at[idxd_v.at[0]], ssems[1]).wait()
    plsc.subcore_barrier()

    @pl.when(s == 0)
    def _():
        pltpu.sync_copy(acc_sp, out_hbm.at[c])


@functools.partial(
    pl.kernel,
    out_type=[
        jax.ShapeDtypeStruct((HEADS, ERP, 128), f32),
        jax.ShapeDtypeStruct((2, 128, 16), f32),
    ],
    mesh=_mesh,
    compiler_params=_sc_params,
    scratch_types=[
        pltpu.VMEM((RT, 128), jnp.int32),
        pltpu.VMEM((RT, 128), jnp.int32),
        pltpu.VMEM((NP,), f32),
        pltpu.VMEM((NP,), f32),
        pltpu.VMEM((NP,), f32),
        pltpu.VMEM((NP,), f32),
        pltpu.VMEM((RT, 128), f32),
        pltpu.VMEM((RT, 128), f32),
        pltpu.VMEM((8, 16), f32),
    ],
)
def _sc_gat_logits(src_hbm, dst_hbm, als0_h, als1_h, als2_h, als3_h,
                   ald0_h, ald1_h, ald2_h, ald3_h, e_hbm, tmax_hbm,
                   idxs_v, idxd_v, ta0, ta1, td0, td1, eb0, eb1, mxv):
    c, s = _ids()

    @pl.when(c == 0)
    def _():
        pltpu.sync_copy(als0_h, ta0)
        pltpu.sync_copy(als1_h, ta1)
        pltpu.sync_copy(ald0_h, td0)
        pltpu.sync_copy(ald1_h, td1)

    @pl.when(c == 1)
    def _():
        pltpu.sync_copy(als2_h, ta0)
        pltpu.sync_copy(als3_h, ta1)
        pltpu.sync_copy(ald2_h, td0)
        pltpu.sync_copy(ald3_h, td1)

    pltpu.sync_copy(src_hbm.at[pl.ds(s * RT, RT)], idxs_v)
    pltpu.sync_copy(dst_hbm.at[pl.ds(s * RT, RT)], idxd_v)

    def row(t, mx):
        def sub(i, mx):
            sv = idxs_v[t, pl.ds(i * 16, 16)]
            dv = idxd_v[t, pl.ds(i * 16, 16)]
            e0 = plsc.load_gather(ta0, [sv]) + plsc.load_gather(td0, [dv])
            e0 = jnp.maximum(e0, 0.2 * e0)
            e1 = plsc.load_gather(ta1, [sv]) + plsc.load_gather(td1, [dv])
            e1 = jnp.maximum(e1, 0.2 * e1)
            eb0[t, pl.ds(i * 16, 16)] = e0
            eb1[t, pl.ds(i * 16, 16)] = e1
            return jnp.maximum(mx, jnp.maximum(e0, e1))

        return lax.fori_loop(0, 8, sub, mx)

    mx = lax.fori_loop(0, RT, row, jnp.full((16,), -3.0e38, f32))
    for r in range(8):
        mxv[r] = mx
    pltpu.sync_copy(eb0, e_hbm.at[2 * c, pl.ds(s * RT, RT)])
    pltpu.sync_copy(eb1, e_hbm.at[2 * c + 1, pl.ds(s * RT, RT)])
    pltpu.sync_copy(mxv, tmax_hbm.at[c, pl.ds(s * 8, 8)])


@functools.partial(
    pl.kernel,
    out_type=[
        jax.ShapeDtypeStruct((2, NP, 128), f32),
        jax.ShapeDtypeStruct((NP,), f32),
        jax.ShapeDtypeStruct((NP,), f32),
        jax.ShapeDtypeStruct((NP,), f32),
        jax.ShapeDtypeStruct((NP,), f32),
    ],
    mesh=_mesh,
    compiler_params=_sc_params,
    scratch_types=[
        pltpu.VMEM((16, 128), jnp.int32),
        pltpu.VMEM((16, 128), jnp.int32),
        pltpu.VMEM((16, 128), f32),
        pltpu.VMEM((16, 128), f32),
        pltpu.VMEM((16,), f32),
        pltpu.VMEM((128, 128), f32),
        pltpu.VMEM((128, 128), f32),
        pltpu.SemaphoreType.DMA,
        pltpu.SemaphoreType.DMA,
        pltpu.SemaphoreType.DMA,
        pltpu.SemaphoreType.DMA,
        pltpu.SemaphoreType.DMA,
        pltpu.SemaphoreType.DMA,
        pltpu.VMEM_SHARED((NP, 128), f32),
        pltpu.VMEM_SHARED((NP,), f32),
        pltpu.VMEM_SHARED((NP,), f32),
    ],
)
def _sc_gat_main(src2_hbm, dst_hbm, e_hbm, m_hbm, h2_hbm,
                 zeros2_hbm, zeros1_hbm,
                 outg_hbm, outs0_h, outs1_h, outs2_h, outs3_h,
                 idxs_v, idxd_v, e0_v, e1_v, m_v, rows0_v, rows1_v,
                 sem0, sem1, ssem0, ssem1, esem0, esem1,
                 acc_sp, s0_sp, s1_sp):
    c, s = _ids()

    @pl.when(s == 0)
    def _():
        pltpu.sync_copy(zeros2_hbm, acc_sp)
        pltpu.sync_copy(zeros1_hbm, s0_sp)
        pltpu.sync_copy(zeros1_hbm, s1_sp)

    pltpu.sync_copy(m_hbm, m_v)
    plsc.subcore_barrier()
    mv = m_v[...]
    base = s * RT
    rows = (rows0_v, rows1_v)
    sems = (sem0, sem1)
    ssems = (ssem0, ssem1)

    def stage(g, sl0):
        row0 = base + g * 8
        pltpu.sync_copy(src2_hbm.at[c, pl.ds(row0, 8)], idxs_v.at[pl.ds(sl0, 8)])
        pltpu.sync_copy(dst_hbm.at[pl.ds(row0, 8)], idxd_v.at[pl.ds(sl0, 8)])
        pltpu.sync_copy(e_hbm.at[2 * c, pl.ds(row0, 8)], e0_v.at[pl.ds(sl0, 8)])
        pltpu.sync_copy(e_hbm.at[2 * c + 1, pl.ds(row0, 8)], e1_v.at[pl.ds(sl0, 8)])

    stage(0, 0)
    pltpu.async_copy(h2_hbm.at[idxs_v.at[0]], rows0_v, sem0)

    def body(ko, carry):
        for b in range(2):
            t = 2 * ko + b
            nb = 1 - b
            g = t // 8
            r = t - g * 8
            par = g % 2
            sl = par * 8 + r

            @pl.when((r == 7) & (t + 1 < RT))
            def _():
                @pl.when(g >= 1)
                def _():
                    for rr in range(8):
                        osl = (1 - par) * 8 + rr
                        pltpu.make_async_copy(
                            e0_v.at[osl], s0_sp.at[idxd_v.at[osl]], esem0).wait()
                        pltpu.make_async_copy(
                            e1_v.at[osl], s1_sp.at[idxd_v.at[osl]], esem1).wait()

                stage(g + 1, (1 - par) * 8)

            tn = t + 1
            gn = tn // 8
            sln = (gn % 2) * 8 + (tn - gn * 8)

            @pl.when(t + 1 < RT)
            def _():
                @pl.when(t >= 1)
                def _():
                    pltpu.make_async_copy(
                        rows[nb], acc_sp.at[idxd_v.at[sl]], ssems[nb]).wait()

                pltpu.async_copy(h2_hbm.at[idxs_v.at[sln]], rows[nb], sems[nb])

            pltpu.make_async_copy(h2_hbm.at[idxs_v.at[sl]], rows[b], sems[b]).wait()

            def expi(i, c2):
                e0_v[sl, pl.ds(i * 16, 16)] = jnp.exp(e0_v[sl, pl.ds(i * 16, 16)] - mv)
                e1_v[sl, pl.ds(i * 16, 16)] = jnp.exp(e1_v[sl, pl.ds(i * 16, 16)] - mv)
                return c2

            lax.fori_loop(0, 8, expi, 0)
            sl16 = jnp.full((16,), sl, jnp.int32)
            rv = rows[b]

            def scale(i, c2):
                for u in range(2):
                    iu = 2 * i + u
                    i16 = jnp.full((16,), iu, jnp.int32)
                    b0 = plsc.load_gather(e0_v, [sl16, i16])
                    b1 = plsc.load_gather(e1_v, [sl16, i16])
                    for j in range(4):
                        rv[iu, pl.ds(j * 16, 16)] = rv[iu, pl.ds(j * 16, 16)] * b0
                    for j in range(4, 8):
                        rv[iu, pl.ds(j * 16, 16)] = rv[iu, pl.ds(j * 16, 16)] * b1
                return c2

            lax.fori_loop(0, 64, scale, 0)

            pltpu.async_copy(e0_v.at[sl], s0_sp.at[idxd_v.at[sl]], esem0,
                             add=True)
            pltpu.async_copy(e1_v.at[sl], s1_sp.at[idxd_v.at[sl]], esem1,
                             add=True)
            pltpu.async_copy(rv, acc_sp.at[idxd_v.at[sl]], ssems[b], add=True)
        return carry

    lax.fori_loop(0, RT // 2, body, 0)
    pltpu.make_async_copy(rows[0], acc_sp.at[idxd_v.at[0]], ssems[0]).wait()
    pltpu.make_async_copy(rows[1], acc_sp.at[idxd_v.at[0]], ssems[1]).wait()
    for rr in range(16):
        pltpu.make_async_copy(e0_v.at[rr], s0_sp.at[idxd_v.at[rr]], esem0).wait()
        pltpu.make_async_copy(e1_v.at[rr], s1_sp.at[idxd_v.at[rr]], esem1).wait()
    plsc.subcore_barrier()

    @pl.when(s == 0)
    def _():
        pltpu.sync_copy(acc_sp, outg_hbm.at[c])

    @pl.when((s == 0) & (c == 0))
    def _():
        pltpu.sync_copy(s0_sp, outs0_h)
        pltpu.sync_copy(s1_sp, outs1_h)

    @pl.when((s == 0) & (c == 1))
    def _():
        pltpu.sync_copy(s0_sp, outs2_h)
        pltpu.sync_copy(s1_sp, outs3_h)


@functools.partial(
    pl.kernel,
    out_type=[jax.ShapeDtypeStruct((NP,), f32),
              jax.ShapeDtypeStruct((NP,), f32)],
    mesh=_mesh,
    compiler_params=_sc_params,
    scratch_types=[
        pltpu.VMEM((RW, 128), jnp.int32),
        pltpu.VMEM((RW, 128), jnp.int32),
        pltpu.VMEM((NP,), f32),
        pltpu.VMEM((128,), f32),
        pltpu.VMEM_SHARED((NP,), f32),
    ],
)
def _sc_gcn2(src_hbm, dst_hbm, zs_hbm, zeros_hbm, out0_hbm, out1_hbm,
             idxs_v, idxd_v, zs_v, upd_v, acc_sp):
    c, s = _ids()
    wid = s * 2 + c

    @pl.when(s == 0)
    def _():
        pltpu.sync_copy(zeros_hbm, acc_sp)

    pltpu.sync_copy(zs_hbm, zs_v)
    pltpu.sync_copy(src_hbm.at[pl.ds(wid * RW, RW)], idxs_v)
    pltpu.sync_copy(dst_hbm.at[pl.ds(wid * RW, RW)], idxd_v)
    plsc.subcore_barrier()

    def body(t, carry):
        def sub(i, cc):
            sv = idxs_v[t, pl.ds(i * 16, 16)]
            upd_v[pl.ds(i * 16, 16)] = plsc.load_gather(zs_v, [sv])
            return cc

        lax.fori_loop(0, 8, sub, 0)
        pltpu.sync_copy(upd_v, acc_sp.at[idxd_v.at[t]], add=True)
        return carry

    lax.fori_loop(0, RW, body, 0)
    plsc.subcore_barrier()

    @pl.when((s == 0) & (c == 0))
    def _():
        pltpu.sync_copy(acc_sp, out0_hbm)

    @pl.when((s == 0) & (c == 1))
    def _():
        pltpu.sync_copy(acc_sp, out1_hbm)


def _tc1_body(x_ref, w_ref, o_ref):
    o_ref[...] = jnp.dot(x_ref[...], w_ref[...], preferred_element_type=f32)


def _tc2_body(dp0_ref, dp1_ref, h1_ref, dinv_ref, hs_ref):
    deg = dp0_ref[...] + dp1_ref[...] + 1.0
    dinv = lax.rsqrt(deg)
    dinv_ref[...] = dinv
    hs_ref[...] = h1_ref[...] * dinv[:, None]


def _tc3_body(agg_ref, hs_ref, dinv_ref, b1_ref, wa_ref, asrc_ref, adst_ref,
              h2_ref, als_ref, ald_ref, eself_ref):
    g1 = jnp.maximum(
        dinv_ref[...][:, None] * (agg_ref[0] + agg_ref[1] + hs_ref[...])
        + b1_ref[...][None, :], 0.0)
    wa = wa_ref[...]
    h2_ref[0] = jnp.dot(g1, wa[:, :128], preferred_element_type=f32)
    h2_ref[1] = jnp.dot(g1, wa[:, 128:], preferred_element_type=f32)
    va = jnp.stack([wa[:, 64 * k:64 * k + 64] @ asrc_ref[k]
                    for k in range(HEADS)], axis=1)
    vd = jnp.stack([wa[:, 64 * k:64 * k + 64] @ adst_ref[k]
                    for k in range(HEADS)], axis=1)
    als = jnp.dot(g1, va, preferred_element_type=f32).T
    ald = jnp.dot(g1, vd, preferred_element_type=f32).T
    als_ref[...] = als
    ald_ref[...] = ald
    es = als + ald
    eself_ref[...] = jnp.maximum(es, 0.2 * es)


def _tc4_body(tmax_ref, eself_ref, m_ref):
    m = jnp.maximum(jnp.max(tmax_ref[...]), jnp.max(eself_ref[...]))
    m_ref[...] = jnp.full((16,), m, f32)


def _tc5_body(outg_ref, outs_ref, eself_ref, m_ref, h2_ref, dinv_ref,
              ba_ref, w3_ref, zs_ref):
    m = m_ref[...][0]
    exs = jnp.exp(eself_ref[...] - m)
    o2 = jnp.zeros((NP, HID), f32)
    for c in range(2):
        for j in range(2):
            k = 2 * c + j
            hk = h2_ref[c][:, 64 * j:64 * j + 64]
            rawk = outg_ref[c][:, 64 * j:64 * j + 64] + exs[k][:, None] * hk
            sk = outs_ref[k] + exs[k]
            o2 = o2 + rawk / (sk[:, None] + 1e-16)
    g2 = jnp.maximum(0.25 * o2 + ba_ref[...][None, :], 0.0)
    z = jnp.dot(g2, w3_ref[...], preferred_element_type=f32)
    zs_ref[...] = z[:, 0] * dinv_ref[...]


def _tc6_body(az0_ref, az1_ref, zs_ref, dinv_ref, b3_ref, o_ref):
    val = dinv_ref[...] * (az0_ref[...] + az1_ref[...] + zs_ref[...]) + b3_ref[0]
    o_ref[...] = (1.0 / (1.0 + jnp.exp(-val)))[:, None]


@jax.jit
def _run(x, edge_index, W1, b1, Wa, a_src, a_dst, ba, W3, b3):
    src = edge_index[0].astype(jnp.int32)
    dst = edge_index[1].astype(jnp.int32)
    pad = jnp.full((EPAD - E,), N, jnp.int32)
    src2d = jnp.concatenate([src, pad]).reshape(ERP, 128)
    dst2d = jnp.concatenate([dst, pad]).reshape(ERP, 128)
    srcs2 = jnp.stack([src2d, src2d + NP])

    xp = jnp.pad(x, ((0, NP - N), (0, 0)))
    zeros1 = jnp.zeros((NP,), f32)
    zeros64 = jnp.zeros((NP, HID), f32)
    zeros128 = jnp.zeros((NP, 128), f32)

    h1 = pl.pallas_call(
        _tc1_body,
        out_shape=jax.ShapeDtypeStruct((NP, HID), f32),
    )(xp, W1)

    dp0, dp1 = _sc_hist(dst2d, zeros1)

    dinv, hs = pl.pallas_call(
        _tc2_body,
        out_shape=[jax.ShapeDtypeStruct((NP,), f32),
                   jax.ShapeDtypeStruct((NP, HID), f32)],
    )(dp0, dp1, h1)

    agg = _sc_gcn1(src2d, dst2d, hs, zeros64)

    h2r, als, ald, eself = pl.pallas_call(
        _tc3_body,
        out_shape=[jax.ShapeDtypeStruct((2, NP, 128), f32),
                   jax.ShapeDtypeStruct((HEADS, NP), f32),
                   jax.ShapeDtypeStruct((HEADS, NP), f32),
                   jax.ShapeDtypeStruct((HEADS, NP), f32)],
    )(agg, hs, dinv, b1, Wa, a_src, a_dst)

    e_edges, tmax = _sc_gat_logits(
        src2d, dst2d, als[0], als[1], als[2], als[3],
        ald[0], ald[1], ald[2], ald[3])

    m16 = pl.pallas_call(
        _tc4_body,
        out_shape=jax.ShapeDtypeStruct((16,), f32),
    )(tmax, eself)

    h2flat = h2r.reshape(2 * NP, 128)
    outg, s0, s1, s2, s3 = _sc_gat_main(srcs2, dst2d, e_edges, m16, h2flat,
                                        zeros128, zeros1)
    outs = jnp.stack([s0, s1, s2, s3])

    zs = pl.pallas_call(
        _tc5_body,
        out_shape=jax.ShapeDtypeStruct((NP,), f32),
    )(outg, outs, eself, m16, h2r, dinv, ba, W3)

    az0, az1 = _sc_gcn2(src2d, dst2d, zs, zeros1)

    out = pl.pallas_call(
        _tc6_body,
        out_shape=jax.ShapeDtypeStruct((NP, 1), f32),
    )(az0, az1, zs, dinv, b3)

    return out[:N]


def kernel(x, edge_index, W1, b1, Wa, a_src, a_dst, ba, W3, b3):
    return _run(x, edge_index, W1, b1, Wa, a_src, a_dst, ba, W3, b3)


# speedup vs baseline: 55.1911x; 1.1279x over previous
import functools

import jax
import jax.numpy as jnp
from jax import lax
from jax.experimental import pallas as pl
from jax.experimental.pallas import tpu as pltpu
from jax.experimental.pallas import tpu_sc as plsc

N = 10000
NP = 10008
E = 320000
ERP = 2560
EPAD = ERP * 128
RW = ERP // 32
RT = ERP // 16
IN_D = 128
HID = 64
HEADS = 4

_mesh = plsc.VectorSubcoreMesh(core_axis_name="c", subcore_axis_name="s")
_sc_params = pltpu.CompilerParams(use_tc_tiling_on_sc=False,
                                 needs_layout_passes=False)
f32 = jnp.float32


def _ids():
    c = lax.axis_index("c")
    s = lax.axis_index("s")
    return c, s


@functools.partial(
    pl.kernel,
    out_type=[jax.ShapeDtypeStruct((NP,), f32),
              jax.ShapeDtypeStruct((NP,), f32)],
    mesh=_mesh,
    compiler_params=_sc_params,
    scratch_types=[
        pltpu.VMEM((RW, 128), jnp.int32),
        pltpu.VMEM((128,), f32),
        pltpu.VMEM_SHARED((NP,), f32),
    ],
)
def _sc_hist(dst_hbm, zeros_hbm, out0_hbm, out1_hbm, idx_v, ones_v, acc_sp):
    c, s = _ids()
    wid = s * 2 + c

    @pl.when(s == 0)
    def _():
        pltpu.sync_copy(zeros_hbm, acc_sp)

    for i in range(8):
        ones_v[pl.ds(i * 16, 16)] = jnp.ones((16,), f32)
    pltpu.sync_copy(dst_hbm.at[pl.ds(wid * RW, RW)], idx_v)
    plsc.subcore_barrier()

    def body(t, carry):
        pltpu.sync_copy(ones_v, acc_sp.at[idx_v.at[t]], add=True)
        return carry

    lax.fori_loop(0, RW, body, 0)
    plsc.subcore_barrier()

    @pl.when((s == 0) & (c == 0))
    def _():
        pltpu.sync_copy(acc_sp, out0_hbm)

    @pl.when((s == 0) & (c == 1))
    def _():
        pltpu.sync_copy(acc_sp, out1_hbm)


@functools.partial(
    pl.kernel,
    out_type=jax.ShapeDtypeStruct((2, NP, HID), f32),
    mesh=_mesh,
    compiler_params=_sc_params,
    scratch_types=[
        pltpu.VMEM((RW, 128), jnp.int32),
        pltpu.VMEM((RW, 128), jnp.int32),
        pltpu.VMEM((128, HID), f32),
        pltpu.VMEM((128, HID), f32),
        pltpu.SemaphoreType.DMA,
        pltpu.SemaphoreType.DMA,
        pltpu.SemaphoreType.DMA,
        pltpu.SemaphoreType.DMA,
        pltpu.VMEM_SHARED((NP, HID), f32),
        pltpu.VMEM_SHARED((NP, HID), f32),
    ],
)
def _sc_gcn1(src_hbm, dst_hbm, hs_hbm, zeros_hbm, out_hbm,
             idxs_v, idxd_v, rows0_v, rows1_v, sem0, sem1, ssem0, ssem1,
             acc_sp, hs_sp):
    c, s = _ids()
    wid = s * 2 + c

    @pl.when(s == 0)
    def _():
        pltpu.sync_copy(zeros_hbm, acc_sp)

    @pl.when(s == 1)
    def _():
        pltpu.sync_copy(hs_hbm, hs_sp)

    pltpu.sync_copy(src_hbm.at[pl.ds(wid * RW, RW)], idxs_v)
    pltpu.sync_copy(dst_hbm.at[pl.ds(wid * RW, RW)], idxd_v)
    plsc.subcore_barrier()

    rows = (rows0_v, rows1_v)
    sems = (sem0, sem1)
    ssems = (ssem0, ssem1)
    pltpu.async_copy(hs_sp.at[idxs_v.at[0]], rows0_v, sem0)

    def body(ko, carry):
        for b in range(2):
            t = 2 * ko + b
            nb = 1 - b

            @pl.when(t + 1 < RW)
            def _():
                @pl.when(t >= 1)
                def _():
                    pltpu.make_async_copy(
                        rows[nb], acc_sp.at[idxd_v.at[t]], ssems[nb]).wait()

                pltpu.async_copy(hs_sp.at[idxs_v.at[t + 1]], rows[nb], sems[nb])

            pltpu.make_async_copy(hs_sp.at[idxs_v.at[t]], rows[b], sems[b]).wait()
            pltpu.async_copy(rows[b], acc_sp.at[idxd_v.at[t]], ssems[b], add=True)
        return carry

    lax.fori_loop(0, RW // 2, body, 0)
    pltpu.make_async_copy(rows[0], acc_sp.at[idxd_v.at[0]], ssems[0]).wait()
    pltpu.make_async_copy(rows[1], acc_sp.at[idxd_v.at[0]], ssems[1]).wait()
    plsc.subcore_barrier()

    @pl.when(s == 0)
    def _():
        pltpu.sync_copy(acc_sp, out_hbm.at[c])


@functools.partial(
    pl.kernel,
    out_type=[
        jax.ShapeDtypeStruct((HEADS, ERP, 128), f32),
        jax.ShapeDtypeStruct((2, 128, 16), f32),
    ],
    mesh=_mesh,
    compiler_params=_sc_params,
    scratch_types=[
        pltpu.VMEM((RT, 128), jnp.int32),
        pltpu.VMEM((RT, 128), jnp.int32),
        pltpu.VMEM((NP,), f32),
        pltpu.VMEM((NP,), f32),
        pltpu.VMEM((NP,), f32),
        pltpu.VMEM((NP,), f32),
        pltpu.VMEM((RT, 128), f32),
        pltpu.VMEM((RT, 128), f32),
        pltpu.VMEM((8, 16), f32),
    ],
)
def _sc_gat_logits(src_hbm, dst_hbm, als0_h, als1_h, als2_h, als3_h,
                   ald0_h, ald1_h, ald2_h, ald3_h, e_hbm, tmax_hbm,
                   idxs_v, idxd_v, ta0, ta1, td0, td1, eb0, eb1, mxv):
    c, s = _ids()

    @pl.when(c == 0)
    def _():
        pltpu.sync_copy(als0_h, ta0)
        pltpu.sync_copy(als1_h, ta1)
        pltpu.sync_copy(ald0_h, td0)
        pltpu.sync_copy(ald1_h, td1)

    @pl.when(c == 1)
    def _():
        pltpu.sync_copy(als2_h, ta0)
        pltpu.sync_copy(als3_h, ta1)
        pltpu.sync_copy(ald2_h, td0)
        pltpu.sync_copy(ald3_h, td1)

    pltpu.sync_copy(src_hbm.at[pl.ds(s * RT, RT)], idxs_v)
    pltpu.sync_copy(dst_hbm.at[pl.ds(s * RT, RT)], idxd_v)

    def row(t, mx):
        def sub(i, mx):
            sv = idxs_v[t, pl.ds(i * 16, 16)]
            dv = idxd_v[t, pl.ds(i * 16, 16)]
            e0 = plsc.load_gather(ta0, [sv]) + plsc.load_gather(td0, [dv])
            e0 = jnp.maximum(e0, 0.2 * e0)
            e1 = plsc.load_gather(ta1, [sv]) + plsc.load_gather(td1, [dv])
            e1 = jnp.maximum(e1, 0.2 * e1)
            eb0[t, pl.ds(i * 16, 16)] = e0
            eb1[t, pl.ds(i * 16, 16)] = e1
            return jnp.maximum(mx, jnp.maximum(e0, e1))

        return lax.fori_loop(0, 8, sub, mx)

    mx = lax.fori_loop(0, RT, row, jnp.full((16,), -3.0e38, f32))
    for r in range(8):
        mxv[r] = mx
    pltpu.sync_copy(eb0, e_hbm.at[2 * c, pl.ds(s * RT, RT)])
    pltpu.sync_copy(eb1, e_hbm.at[2 * c + 1, pl.ds(s * RT, RT)])
    pltpu.sync_copy(mxv, tmax_hbm.at[c, pl.ds(s * 8, 8)])


@functools.partial(
    pl.kernel,
    out_type=[
        jax.ShapeDtypeStruct((2, NP, 128), f32),
        jax.ShapeDtypeStruct((NP,), f32),
        jax.ShapeDtypeStruct((NP,), f32),
        jax.ShapeDtypeStruct((NP,), f32),
        jax.ShapeDtypeStruct((NP,), f32),
    ],
    mesh=_mesh,
    compiler_params=_sc_params,
    scratch_types=[
        pltpu.VMEM((16, 128), jnp.int32),
        pltpu.VMEM((16, 128), jnp.int32),
        pltpu.VMEM((16, 128), f32),
        pltpu.VMEM((16, 128), f32),
        pltpu.VMEM((128, 128), f32),
        pltpu.VMEM((128, 128), f32),
        pltpu.SemaphoreType.DMA,
        pltpu.SemaphoreType.DMA,
        pltpu.SemaphoreType.DMA,
        pltpu.SemaphoreType.DMA,
        pltpu.SemaphoreType.DMA,
        pltpu.SemaphoreType.DMA,
        pltpu.VMEM_SHARED((NP, 128), f32),
        pltpu.VMEM_SHARED((NP,), f32),
        pltpu.VMEM_SHARED((NP,), f32),
    ],
)
def _sc_gat_main(src2_hbm, dst_hbm, e_hbm, h2_hbm,
                 zeros2_hbm, zeros1_hbm,
                 outg_hbm, outs0_h, outs1_h, outs2_h, outs3_h,
                 idxs_v, idxd_v, e0_v, e1_v, rows0_v, rows1_v,
                 sem0, sem1, ssem0, ssem1, esem0, esem1,
                 acc_sp, s0_sp, s1_sp):
    c, s = _ids()

    @pl.when(s == 0)
    def _():
        pltpu.sync_copy(zeros2_hbm, acc_sp)
        pltpu.sync_copy(zeros1_hbm, s0_sp)
        pltpu.sync_copy(zeros1_hbm, s1_sp)

    plsc.subcore_barrier()
    base = s * RT
    rows = (rows0_v, rows1_v)
    sems = (sem0, sem1)
    ssems = (ssem0, ssem1)

    def stage(g, sl0):
        row0 = base + g * 8
        pltpu.sync_copy(src2_hbm.at[c, pl.ds(row0, 8)], idxs_v.at[pl.ds(sl0, 8)])
        pltpu.sync_copy(dst_hbm.at[pl.ds(row0, 8)], idxd_v.at[pl.ds(sl0, 8)])
        pltpu.sync_copy(e_hbm.at[2 * c, pl.ds(row0, 8)], e0_v.at[pl.ds(sl0, 8)])
        pltpu.sync_copy(e_hbm.at[2 * c + 1, pl.ds(row0, 8)], e1_v.at[pl.ds(sl0, 8)])

    stage(0, 0)
    pltpu.async_copy(h2_hbm.at[idxs_v.at[0]], rows0_v, sem0)

    def body(ko, carry):
        for b in range(2):
            t = 2 * ko + b
            nb = 1 - b
            g = t // 8
            r = t - g * 8
            par = g % 2
            sl = par * 8 + r

            @pl.when((r == 7) & (t + 1 < RT))
            def _():
                @pl.when(g >= 1)
                def _():
                    for rr in range(8):
                        osl = (1 - par) * 8 + rr
                        pltpu.make_async_copy(
                            e0_v.at[osl], s0_sp.at[idxd_v.at[osl]], esem0).wait()
                        pltpu.make_async_copy(
                            e1_v.at[osl], s1_sp.at[idxd_v.at[osl]], esem1).wait()

                stage(g + 1, (1 - par) * 8)

            tn = t + 1
            gn = tn // 8
            sln = (gn % 2) * 8 + (tn - gn * 8)

            @pl.when(t + 1 < RT)
            def _():
                @pl.when(t >= 1)
                def _():
                    pltpu.make_async_copy(
                        rows[nb], acc_sp.at[idxd_v.at[sl]], ssems[nb]).wait()

                pltpu.async_copy(h2_hbm.at[idxs_v.at[sln]], rows[nb], sems[nb])

            pltpu.make_async_copy(h2_hbm.at[idxs_v.at[sl]], rows[b], sems[b]).wait()
            sl16 = jnp.full((16,), sl, jnp.int32)
            rv = rows[b]

            def scale(i, c2):
                for u in range(2):
                    iu = 2 * i + u
                    i16 = jnp.full((16,), iu, jnp.int32)
                    b0 = plsc.load_gather(e0_v, [sl16, i16])
                    b1 = plsc.load_gather(e1_v, [sl16, i16])
                    for j in range(4):
                        rv[iu, pl.ds(j * 16, 16)] = rv[iu, pl.ds(j * 16, 16)] * b0
                    for j in range(4, 8):
                        rv[iu, pl.ds(j * 16, 16)] = rv[iu, pl.ds(j * 16, 16)] * b1
                return c2

            lax.fori_loop(0, 64, scale, 0)

            pltpu.async_copy(e0_v.at[sl], s0_sp.at[idxd_v.at[sl]], esem0,
                             add=True)
            pltpu.async_copy(e1_v.at[sl], s1_sp.at[idxd_v.at[sl]], esem1,
                             add=True)
            pltpu.async_copy(rv, acc_sp.at[idxd_v.at[sl]], ssems[b], add=True)
        return carry

    lax.fori_loop(0, RT // 2, body, 0)
    pltpu.make_async_copy(rows[0], acc_sp.at[idxd_v.at[0]], ssems[0]).wait()
    pltpu.make_async_copy(rows[1], acc_sp.at[idxd_v.at[0]], ssems[1]).wait()
    for rr in range(16):
        pltpu.make_async_copy(e0_v.at[rr], s0_sp.at[idxd_v.at[rr]], esem0).wait()
        pltpu.make_async_copy(e1_v.at[rr], s1_sp.at[idxd_v.at[rr]], esem1).wait()
    plsc.subcore_barrier()

    @pl.when(s == 0)
    def _():
        pltpu.sync_copy(acc_sp, outg_hbm.at[c])

    @pl.when((s == 0) & (c == 0))
    def _():
        pltpu.sync_copy(s0_sp, outs0_h)
        pltpu.sync_copy(s1_sp, outs1_h)

    @pl.when((s == 0) & (c == 1))
    def _():
        pltpu.sync_copy(s0_sp, outs2_h)
        pltpu.sync_copy(s1_sp, outs3_h)


@functools.partial(
    pl.kernel,
    out_type=[jax.ShapeDtypeStruct((NP,), f32),
              jax.ShapeDtypeStruct((NP,), f32)],
    mesh=_mesh,
    compiler_params=_sc_params,
    scratch_types=[
        pltpu.VMEM((RW, 128), jnp.int32),
        pltpu.VMEM((RW, 128), jnp.int32),
        pltpu.VMEM((NP,), f32),
        pltpu.VMEM((128,), f32),
        pltpu.VMEM_SHARED((NP,), f32),
    ],
)
def _sc_gcn2(src_hbm, dst_hbm, zs_hbm, zeros_hbm, out0_hbm, out1_hbm,
             idxs_v, idxd_v, zs_v, upd_v, acc_sp):
    c, s = _ids()
    wid = s * 2 + c

    @pl.when(s == 0)
    def _():
        pltpu.sync_copy(zeros_hbm, acc_sp)

    pltpu.sync_copy(zs_hbm, zs_v)
    pltpu.sync_copy(src_hbm.at[pl.ds(wid * RW, RW)], idxs_v)
    pltpu.sync_copy(dst_hbm.at[pl.ds(wid * RW, RW)], idxd_v)
    plsc.subcore_barrier()

    def body(t, carry):
        def sub(i, cc):
            sv = idxs_v[t, pl.ds(i * 16, 16)]
            upd_v[pl.ds(i * 16, 16)] = plsc.load_gather(zs_v, [sv])
            return cc

        lax.fori_loop(0, 8, sub, 0)
        pltpu.sync_copy(upd_v, acc_sp.at[idxd_v.at[t]], add=True)
        return carry

    lax.fori_loop(0, RW, body, 0)
    plsc.subcore_barrier()

    @pl.when((s == 0) & (c == 0))
    def _():
        pltpu.sync_copy(acc_sp, out0_hbm)

    @pl.when((s == 0) & (c == 1))
    def _():
        pltpu.sync_copy(acc_sp, out1_hbm)


def _tc1_body(x_ref, w_ref, o_ref):
    o_ref[...] = jnp.dot(x_ref[...], w_ref[...], preferred_element_type=f32)


def _tc2_body(dp0_ref, dp1_ref, h1_ref, dinv_ref, hs_ref):
    deg = dp0_ref[...] + dp1_ref[...] + 1.0
    dinv = lax.rsqrt(deg)
    dinv_ref[...] = dinv
    hs_ref[...] = h1_ref[...] * dinv[:, None]


def _tc3_body(agg_ref, hs_ref, dinv_ref, b1_ref, wa_ref, asrc_ref, adst_ref,
              h2_ref, als_ref, ald_ref, eself_ref):
    g1 = jnp.maximum(
        dinv_ref[...][:, None] * (agg_ref[0] + agg_ref[1] + hs_ref[...])
        + b1_ref[...][None, :], 0.0)
    wa = wa_ref[...]
    h2_ref[0] = jnp.dot(g1, wa[:, :128], preferred_element_type=f32)
    h2_ref[1] = jnp.dot(g1, wa[:, 128:], preferred_element_type=f32)
    va = jnp.stack([wa[:, 64 * k:64 * k + 64] @ asrc_ref[k]
                    for k in range(HEADS)], axis=1)
    vd = jnp.stack([wa[:, 64 * k:64 * k + 64] @ adst_ref[k]
                    for k in range(HEADS)], axis=1)
    als = jnp.dot(g1, va, preferred_element_type=f32).T
    ald = jnp.dot(g1, vd, preferred_element_type=f32).T
    als_ref[...] = als
    ald_ref[...] = ald
    es = als + ald
    eself_ref[...] = jnp.maximum(es, 0.2 * es)


def _tc4_body(tmax_ref, eself_ref, e_ref, exs_ref, ex_ref):
    m = jnp.maximum(jnp.max(tmax_ref[...]), jnp.max(eself_ref[...]))
    exs_ref[...] = jnp.exp(eself_ref[...] - m)
    ex_ref[...] = jnp.exp(e_ref[...] - m)


def _tc5_body(outg_ref, outs_ref, exself_ref, h2_ref, dinv_ref,
              ba_ref, w3_ref, zs_ref):
    exs = exself_ref[...]
    o2 = jnp.zeros((NP, HID), f32)
    for c in range(2):
        for j in range(2):
            k = 2 * c + j
            hk = h2_ref[c][:, 64 * j:64 * j + 64]
            rawk = outg_ref[c][:, 64 * j:64 * j + 64] + exs[k][:, None] * hk
            sk = outs_ref[k] + exs[k]
            o2 = o2 + rawk / (sk[:, None] + 1e-16)
    g2 = jnp.maximum(0.25 * o2 + ba_ref[...][None, :], 0.0)
    z = jnp.dot(g2, w3_ref[...], preferred_element_type=f32)
    zs_ref[...] = z[:, 0] * dinv_ref[...]


def _tc6_body(az0_ref, az1_ref, zs_ref, dinv_ref, b3_ref, o_ref):
    val = dinv_ref[...] * (az0_ref[...] + az1_ref[...] + zs_ref[...]) + b3_ref[0]
    o_ref[...] = (1.0 / (1.0 + jnp.exp(-val)))[:, None]


@jax.jit
def _run(x, edge_index, W1, b1, Wa, a_src, a_dst, ba, W3, b3):
    src = edge_index[0].astype(jnp.int32)
    dst = edge_index[1].astype(jnp.int32)
    pad = jnp.full((EPAD - E,), N, jnp.int32)
    src2d = jnp.concatenate([src, pad]).reshape(ERP, 128)
    dst2d = jnp.concatenate([dst, pad]).reshape(ERP, 128)
    srcs2 = jnp.stack([src2d, src2d + NP])

    xp = jnp.pad(x, ((0, NP - N), (0, 0)))
    zeros1 = jnp.zeros((NP,), f32)
    zeros64 = jnp.zeros((NP, HID), f32)
    zeros128 = jnp.zeros((NP, 128), f32)

    h1 = pl.pallas_call(
        _tc1_body,
        out_shape=jax.ShapeDtypeStruct((NP, HID), f32),
    )(xp, W1)

    dp0, dp1 = _sc_hist(dst2d, zeros1)

    dinv, hs = pl.pallas_call(
        _tc2_body,
        out_shape=[jax.ShapeDtypeStruct((NP,), f32),
                   jax.ShapeDtypeStruct((NP, HID), f32)],
    )(dp0, dp1, h1)

    agg = _sc_gcn1(src2d, dst2d, hs, zeros64)

    h2r, als, ald, eself = pl.pallas_call(
        _tc3_body,
        out_shape=[jax.ShapeDtypeStruct((2, NP, 128), f32),
                   jax.ShapeDtypeStruct((HEADS, NP), f32),
                   jax.ShapeDtypeStruct((HEADS, NP), f32),
                   jax.ShapeDtypeStruct((HEADS, NP), f32)],
    )(agg, hs, dinv, b1, Wa, a_src, a_dst)

    e_edges, tmax = _sc_gat_logits(
        src2d, dst2d, als[0], als[1], als[2], als[3],
        ald[0], ald[1], ald[2], ald[3])

    exself, ex_edges = pl.pallas_call(
        _tc4_body,
        out_shape=[jax.ShapeDtypeStruct((HEADS, NP), f32),
                   jax.ShapeDtypeStruct((HEADS, ERP, 128), f32)],
    )(tmax, eself, e_edges)

    h2flat = h2r.reshape(2 * NP, 128)
    outg, s0, s1, s2, s3 = _sc_gat_main(srcs2, dst2d, ex_edges, h2flat,
                                        zeros128, zeros1)
    outs = jnp.stack([s0, s1, s2, s3])

    zs = pl.pallas_call(
        _tc5_body,
        out_shape=jax.ShapeDtypeStruct((NP,), f32),
    )(outg, outs, exself, h2r, dinv, ba, W3)

    az0, az1 = _sc_gcn2(src2d, dst2d, zs, zeros1)

    out = pl.pallas_call(
        _tc6_body,
        out_shape=jax.ShapeDtypeStruct((NP, 1), f32),
    )(az0, az1, zs, dinv, b3)

    return out[:N]


def kernel(x, edge_index, W1, b1, Wa, a_src, a_dst, ba, W3, b3):
    return _run(x, edge_index, W1, b1, Wa, a_src, a_dst, ba, W3, b3)


# speedup vs baseline: 61.5649x; 1.1155x over previous
import functools

import jax
import jax.numpy as jnp
from jax import lax
from jax.experimental import pallas as pl
from jax.experimental.pallas import tpu as pltpu
from jax.experimental.pallas import tpu_sc as plsc

N = 10000
NP = 10008
E = 320000
ERP = 2560
EPAD = ERP * 128
RW = ERP // 32
RT = ERP // 16
IN_D = 128
HID = 64
HEADS = 4

_mesh = plsc.VectorSubcoreMesh(core_axis_name="c", subcore_axis_name="s")
_sc_params = pltpu.CompilerParams(use_tc_tiling_on_sc=False,
                                 needs_layout_passes=False)
f32 = jnp.float32


def _ids():
    c = lax.axis_index("c")
    s = lax.axis_index("s")
    return c, s


@functools.partial(
    pl.kernel,
    out_type=[jax.ShapeDtypeStruct((NP,), f32),
              jax.ShapeDtypeStruct((NP,), f32)],
    mesh=_mesh,
    compiler_params=_sc_params,
    scratch_types=[
        pltpu.VMEM((RW, 128), jnp.int32),
        pltpu.VMEM((128,), f32),
        pltpu.VMEM_SHARED((NP,), f32),
    ],
)
def _sc_hist(dst_hbm, zeros_hbm, out0_hbm, out1_hbm, idx_v, ones_v, acc_sp):
    c, s = _ids()
    wid = s * 2 + c

    @pl.when(s == 0)
    def _():
        pltpu.sync_copy(zeros_hbm, acc_sp)

    for i in range(8):
        ones_v[pl.ds(i * 16, 16)] = jnp.ones((16,), f32)
    pltpu.sync_copy(dst_hbm.at[pl.ds(wid * RW, RW)], idx_v)
    plsc.subcore_barrier()

    def body(t, carry):
        pltpu.sync_copy(ones_v, acc_sp.at[idx_v.at[t]], add=True)
        return carry

    lax.fori_loop(0, RW, body, 0)
    plsc.subcore_barrier()

    @pl.when((s == 0) & (c == 0))
    def _():
        pltpu.sync_copy(acc_sp, out0_hbm)

    @pl.when((s == 0) & (c == 1))
    def _():
        pltpu.sync_copy(acc_sp, out1_hbm)


@functools.partial(
    pl.kernel,
    out_type=jax.ShapeDtypeStruct((2, NP, HID), f32),
    mesh=_mesh,
    compiler_params=_sc_params,
    scratch_types=[
        pltpu.VMEM((RW, 128), jnp.int32),
        pltpu.VMEM((RW, 128), jnp.int32),
        pltpu.VMEM((128, HID), f32),
        pltpu.VMEM((128, HID), f32),
        pltpu.SemaphoreType.DMA,
        pltpu.SemaphoreType.DMA,
        pltpu.SemaphoreType.DMA,
        pltpu.SemaphoreType.DMA,
        pltpu.VMEM_SHARED((NP, HID), f32),
        pltpu.VMEM_SHARED((NP, HID), f32),
    ],
)
def _sc_gcn1(src_hbm, dst_hbm, hs_hbm, zeros_hbm, out_hbm,
             idxs_v, idxd_v, rows0_v, rows1_v, sem0, sem1, ssem0, ssem1,
             acc_sp, hs_sp):
    c, s = _ids()
    wid = s * 2 + c

    @pl.when(s == 0)
    def _():
        pltpu.sync_copy(zeros_hbm, acc_sp)

    @pl.when(s == 1)
    def _():
        pltpu.sync_copy(hs_hbm, hs_sp)

    pltpu.sync_copy(src_hbm.at[pl.ds(wid * RW, RW)], idxs_v)
    pltpu.sync_copy(dst_hbm.at[pl.ds(wid * RW, RW)], idxd_v)
    plsc.subcore_barrier()

    rows = (rows0_v, rows1_v)
    sems = (sem0, sem1)
    ssems = (ssem0, ssem1)
    pltpu.async_copy(hs_sp.at[idxs_v.at[0]], rows0_v, sem0)

    def body(ko, carry):
        for b in range(2):
            t = 2 * ko + b
            nb = 1 - b

            @pl.when(t + 1 < RW)
            def _():
                @pl.when(t >= 1)
                def _():
                    pltpu.make_async_copy(
                        rows[nb], acc_sp.at[idxd_v.at[t]], ssems[nb]).wait()

                pltpu.async_copy(hs_sp.at[idxs_v.at[t + 1]], rows[nb], sems[nb])

            pltpu.make_async_copy(hs_sp.at[idxs_v.at[t]], rows[b], sems[b]).wait()
            pltpu.async_copy(rows[b], acc_sp.at[idxd_v.at[t]], ssems[b], add=True)
        return carry

    lax.fori_loop(0, RW // 2, body, 0)
    pltpu.make_async_copy(rows[0], acc_sp.at[idxd_v.at[0]], ssems[0]).wait()
    pltpu.make_async_copy(rows[1], acc_sp.at[idxd_v.at[0]], ssems[1]).wait()
    plsc.subcore_barrier()

    @pl.when(s == 0)
    def _():
        pltpu.sync_copy(acc_sp, out_hbm.at[c])


@functools.partial(
    pl.kernel,
    out_type=[
        jax.ShapeDtypeStruct((HEADS, ERP, 128), f32),
        jax.ShapeDtypeStruct((2, 128, 16), f32),
    ],
    mesh=_mesh,
    compiler_params=_sc_params,
    scratch_types=[
        pltpu.VMEM((RT, 128), jnp.int32),
        pltpu.VMEM((RT, 128), jnp.int32),
        pltpu.VMEM((NP,), f32),
        pltpu.VMEM((NP,), f32),
        pltpu.VMEM((NP,), f32),
        pltpu.VMEM((NP,), f32),
        pltpu.VMEM((RT, 128), f32),
        pltpu.VMEM((RT, 128), f32),
        pltpu.VMEM((8, 16), f32),
    ],
)
def _sc_gat_logits(src_hbm, dst_hbm, als0_h, als1_h, als2_h, als3_h,
                   ald0_h, ald1_h, ald2_h, ald3_h, e_hbm, tmax_hbm,
                   idxs_v, idxd_v, ta0, ta1, td0, td1, eb0, eb1, mxv):
    c, s = _ids()

    @pl.when(c == 0)
    def _():
        pltpu.sync_copy(als0_h, ta0)
        pltpu.sync_copy(als1_h, ta1)
        pltpu.sync_copy(ald0_h, td0)
        pltpu.sync_copy(ald1_h, td1)

    @pl.when(c == 1)
    def _():
        pltpu.sync_copy(als2_h, ta0)
        pltpu.sync_copy(als3_h, ta1)
        pltpu.sync_copy(ald2_h, td0)
        pltpu.sync_copy(ald3_h, td1)

    pltpu.sync_copy(src_hbm.at[pl.ds(s * RT, RT)], idxs_v)
    pltpu.sync_copy(dst_hbm.at[pl.ds(s * RT, RT)], idxd_v)

    def row(t, mx):
        def sub(i, mx):
            sv = idxs_v[t, pl.ds(i * 16, 16)]
            dv = idxd_v[t, pl.ds(i * 16, 16)]
            e0 = plsc.load_gather(ta0, [sv]) + plsc.load_gather(td0, [dv])
            e0 = jnp.maximum(e0, 0.2 * e0)
            e1 = plsc.load_gather(ta1, [sv]) + plsc.load_gather(td1, [dv])
            e1 = jnp.maximum(e1, 0.2 * e1)
            eb0[t, pl.ds(i * 16, 16)] = e0
            eb1[t, pl.ds(i * 16, 16)] = e1
            return jnp.maximum(mx, jnp.maximum(e0, e1))

        return lax.fori_loop(0, 8, sub, mx)

    mx = lax.fori_loop(0, RT, row, jnp.full((16,), -3.0e38, f32))
    for r in range(8):
        mxv[r] = mx
    pltpu.sync_copy(eb0, e_hbm.at[2 * c, pl.ds(s * RT, RT)])
    pltpu.sync_copy(eb1, e_hbm.at[2 * c + 1, pl.ds(s * RT, RT)])
    pltpu.sync_copy(mxv, tmax_hbm.at[c, pl.ds(s * 8, 8)])


@functools.partial(
    pl.kernel,
    out_type=[
        jax.ShapeDtypeStruct((HEADS, NP, HID), f32),
        jax.ShapeDtypeStruct((NP,), f32),
        jax.ShapeDtypeStruct((NP,), f32),
        jax.ShapeDtypeStruct((NP,), f32),
        jax.ShapeDtypeStruct((NP,), f32),
    ],
    mesh=_mesh,
    compiler_params=_sc_params,
    scratch_types=[
        pltpu.VMEM((16, 128), jnp.int32),
        pltpu.VMEM((16, 128), jnp.int32),
        pltpu.VMEM((16, 128), f32),
        pltpu.VMEM((128, HID), f32),
        pltpu.VMEM((128, HID), f32),
        pltpu.SemaphoreType.DMA,
        pltpu.SemaphoreType.DMA,
        pltpu.SemaphoreType.DMA,
        pltpu.SemaphoreType.DMA,
        pltpu.SemaphoreType.DMA,
        pltpu.VMEM_SHARED((NP, HID), f32),
        pltpu.VMEM_SHARED((NP, HID), f32),
        pltpu.VMEM_SHARED((NP,), f32),
    ],
)
def _sc_gat_main(src_hbm, dst_hbm, e_hbm, h2h_hbm,
                 zeros64_hbm, zeros1_hbm,
                 outg_hbm, outs0_h, outs1_h, outs2_h, outs3_h,
                 idxs_v, idxd_v, e_v, rows0_v, rows1_v,
                 sem0, sem1, ssem0, ssem1, esem,
                 table_sp, acc_sp, s_sp):
    c, s = _ids()
    base = s * RT
    rows = (rows0_v, rows1_v)
    sems = (sem0, sem1)
    ssems = (ssem0, ssem1)

    for j in range(2):
        k2 = 2 * c + j

        @pl.when(s == 0)
        def _():
            pltpu.sync_copy(zeros64_hbm, acc_sp)
            pltpu.sync_copy(zeros1_hbm, s_sp)

        @pl.when(s == 1)
        def _():
            pltpu.sync_copy(h2h_hbm.at[k2], table_sp)

        def stage(g, sl0):
            row0 = base + g * 8
            pltpu.sync_copy(src_hbm.at[pl.ds(row0, 8)],
                            idxs_v.at[pl.ds(sl0, 8)])
            pltpu.sync_copy(dst_hbm.at[pl.ds(row0, 8)],
                            idxd_v.at[pl.ds(sl0, 8)])
            pltpu.sync_copy(e_hbm.at[k2, pl.ds(row0, 8)],
                            e_v.at[pl.ds(sl0, 8)])

        stage(0, 0)
        plsc.subcore_barrier()
        pltpu.async_copy(table_sp.at[idxs_v.at[0]], rows0_v, sem0)

        def body(ko, carry):
            for b in range(2):
                t = 2 * ko + b
                nb = 1 - b
                g = t // 8
                r = t - g * 8
                par = g % 2
                sl = par * 8 + r

                @pl.when((r == 7) & (t + 1 < RT))
                def _():
                    @pl.when(g >= 1)
                    def _():
                        for rr in range(8):
                            osl = (1 - par) * 8 + rr
                            pltpu.make_async_copy(
                                e_v.at[osl], s_sp.at[idxd_v.at[osl]],
                                esem).wait()

                    stage(g + 1, (1 - par) * 8)

                tn = t + 1
                gn = tn // 8
                sln = (gn % 2) * 8 + (tn - gn * 8)

                @pl.when(t + 1 < RT)
                def _():
                    @pl.when(t >= 1)
                    def _():
                        pltpu.make_async_copy(
                            rows[nb], acc_sp.at[idxd_v.at[sl]],
                            ssems[nb]).wait()

                    pltpu.async_copy(table_sp.at[idxs_v.at[sln]],
                                     rows[nb], sems[nb])

                pltpu.make_async_copy(table_sp.at[idxs_v.at[sl]],
                                      rows[b], sems[b]).wait()
                sl16 = jnp.full((16,), sl, jnp.int32)
                rv = rows[b]

                def scale(i, c2):
                    for u in range(2):
                        iu = 2 * i + u
                        i16 = jnp.full((16,), iu, jnp.int32)
                        b0 = plsc.load_gather(e_v, [sl16, i16])
                        for jj in range(4):
                            rv[iu, pl.ds(jj * 16, 16)] = (
                                rv[iu, pl.ds(jj * 16, 16)] * b0)
                    return c2

                lax.fori_loop(0, 64, scale, 0)
                pltpu.async_copy(e_v.at[sl], s_sp.at[idxd_v.at[sl]], esem,
                                 add=True)
                pltpu.async_copy(rv, acc_sp.at[idxd_v.at[sl]], ssems[b],
                                 add=True)
            return carry

        lax.fori_loop(0, RT // 2, body, 0)
        pltpu.make_async_copy(rows[0], acc_sp.at[idxd_v.at[0]], ssems[0]).wait()
        pltpu.make_async_copy(rows[1], acc_sp.at[idxd_v.at[0]], ssems[1]).wait()
        for rr in range(16):
            pltpu.make_async_copy(e_v.at[rr], s_sp.at[idxd_v.at[rr]],
                                  esem).wait()
        plsc.subcore_barrier()

        @pl.when(s == 0)
        def _():
            pltpu.sync_copy(acc_sp, outg_hbm.at[k2])

        if j == 0:
            @pl.when((s == 0) & (c == 0))
            def _():
                pltpu.sync_copy(s_sp, outs0_h)

            @pl.when((s == 0) & (c == 1))
            def _():
                pltpu.sync_copy(s_sp, outs2_h)
        else:
            @pl.when((s == 0) & (c == 0))
            def _():
                pltpu.sync_copy(s_sp, outs1_h)

            @pl.when((s == 0) & (c == 1))
            def _():
                pltpu.sync_copy(s_sp, outs3_h)

        plsc.subcore_barrier()


@functools.partial(
    pl.kernel,
    out_type=[jax.ShapeDtypeStruct((NP,), f32),
              jax.ShapeDtypeStruct((NP,), f32)],
    mesh=_mesh,
    compiler_params=_sc_params,
    scratch_types=[
        pltpu.VMEM((RW, 128), jnp.int32),
        pltpu.VMEM((RW, 128), jnp.int32),
        pltpu.VMEM((NP,), f32),
        pltpu.VMEM((128,), f32),
        pltpu.VMEM_SHARED((NP,), f32),
    ],
)
def _sc_gcn2(src_hbm, dst_hbm, zs_hbm, zeros_hbm, out0_hbm, out1_hbm,
             idxs_v, idxd_v, zs_v, upd_v, acc_sp):
    c, s = _ids()
    wid = s * 2 + c

    @pl.when(s == 0)
    def _():
        pltpu.sync_copy(zeros_hbm, acc_sp)

    pltpu.sync_copy(zs_hbm, zs_v)
    pltpu.sync_copy(src_hbm.at[pl.ds(wid * RW, RW)], idxs_v)
    pltpu.sync_copy(dst_hbm.at[pl.ds(wid * RW, RW)], idxd_v)
    plsc.subcore_barrier()

    def body(t, carry):
        def sub(i, cc):
            sv = idxs_v[t, pl.ds(i * 16, 16)]
            upd_v[pl.ds(i * 16, 16)] = plsc.load_gather(zs_v, [sv])
            return cc

        lax.fori_loop(0, 8, sub, 0)
        pltpu.sync_copy(upd_v, acc_sp.at[idxd_v.at[t]], add=True)
        return carry

    lax.fori_loop(0, RW, body, 0)
    plsc.subcore_barrier()

    @pl.when((s == 0) & (c == 0))
    def _():
        pltpu.sync_copy(acc_sp, out0_hbm)

    @pl.when((s == 0) & (c == 1))
    def _():
        pltpu.sync_copy(acc_sp, out1_hbm)


def _tc1_body(x_ref, w_ref, o_ref):
    o_ref[...] = jnp.dot(x_ref[...], w_ref[...], preferred_element_type=f32)


def _tc2_body(dp0_ref, dp1_ref, h1_ref, dinv_ref, hs_ref):
    deg = dp0_ref[...] + dp1_ref[...] + 1.0
    dinv = lax.rsqrt(deg)
    dinv_ref[...] = dinv
    hs_ref[...] = h1_ref[...] * dinv[:, None]


def _tc3_body(agg_ref, hs_ref, dinv_ref, b1_ref, wa_ref, asrc_ref, adst_ref,
              h2_ref, als_ref, ald_ref, eself_ref):
    g1 = jnp.maximum(
        dinv_ref[...][:, None] * (agg_ref[0] + agg_ref[1] + hs_ref[...])
        + b1_ref[...][None, :], 0.0)
    wa = wa_ref[...]
    for k in range(HEADS):
        h2_ref[k] = jnp.dot(g1, wa[:, 64 * k:64 * k + 64],
                            preferred_element_type=f32)
    va = jnp.stack([wa[:, 64 * k:64 * k + 64] @ asrc_ref[k]
                    for k in range(HEADS)], axis=1)
    vd = jnp.stack([wa[:, 64 * k:64 * k + 64] @ adst_ref[k]
                    for k in range(HEADS)], axis=1)
    als = jnp.dot(g1, va, preferred_element_type=f32).T
    ald = jnp.dot(g1, vd, preferred_element_type=f32).T
    als_ref[...] = als
    ald_ref[...] = ald
    es = als + ald
    eself_ref[...] = jnp.maximum(es, 0.2 * es)


def _tc4_body(tmax_ref, eself_ref, e_ref, exs_ref, ex_ref):
    m = jnp.maximum(jnp.max(tmax_ref[...]), jnp.max(eself_ref[...]))
    exs_ref[...] = jnp.exp(eself_ref[...] - m).T
    ex_ref[...] = jnp.exp(e_ref[...] - m)


_TC5_B = 3336


def _tc5_body(outg_ref, outs_ref, exself_ref, h2_ref, dinv_ref,
              ba_ref, w3_ref, zs_ref):
    exs = exself_ref[...]
    o2 = jnp.zeros((_TC5_B, HID), f32)
    for k in range(HEADS):
        hk = h2_ref[k]
        ek = exs[:, k]
        rawk = outg_ref[k] + ek[:, None] * hk
        sk = outs_ref[:, k] + ek
        o2 = o2 + rawk / (sk[:, None] + 1e-16)
    g2 = jnp.maximum(0.25 * o2 + ba_ref[...][None, :], 0.0)
    z = jnp.dot(g2, w3_ref[...], preferred_element_type=f32)
    zs_ref[...] = z * dinv_ref[...]


def _tc6_body(az0_ref, az1_ref, zs_ref, dinv_ref, b3_ref, o_ref):
    val = dinv_ref[...] * (az0_ref[...] + az1_ref[...] + zs_ref[...]) + b3_ref[0]
    o_ref[...] = (1.0 / (1.0 + jnp.exp(-val)))[:, None]


@jax.jit
def _run(x, edge_index, W1, b1, Wa, a_src, a_dst, ba, W3, b3):
    src = edge_index[0].astype(jnp.int32)
    dst = edge_index[1].astype(jnp.int32)
    pad = jnp.full((EPAD - E,), N, jnp.int32)
    src2d = jnp.concatenate([src, pad]).reshape(ERP, 128)
    dst2d = jnp.concatenate([dst, pad]).reshape(ERP, 128)

    xp = jnp.pad(x, ((0, NP - N), (0, 0)))
    zeros1 = jnp.zeros((NP,), f32)
    zeros64 = jnp.zeros((NP, HID), f32)

    h1 = pl.pallas_call(
        _tc1_body,
        out_shape=jax.ShapeDtypeStruct((NP, HID), f32),
    )(xp, W1)

    dp0, dp1 = _sc_hist(dst2d, zeros1)

    dinv, hs = pl.pallas_call(
        _tc2_body,
        out_shape=[jax.ShapeDtypeStruct((NP,), f32),
                   jax.ShapeDtypeStruct((NP, HID), f32)],
    )(dp0, dp1, h1)

    agg = _sc_gcn1(src2d, dst2d, hs, zeros64)

    h2h, als, ald, eself = pl.pallas_call(
        _tc3_body,
        out_shape=[jax.ShapeDtypeStruct((HEADS, NP, HID), f32),
                   jax.ShapeDtypeStruct((HEADS, NP), f32),
                   jax.ShapeDtypeStruct((HEADS, NP), f32),
                   jax.ShapeDtypeStruct((HEADS, NP), f32)],
    )(agg, hs, dinv, b1, Wa, a_src, a_dst)

    e_edges, tmax = _sc_gat_logits(
        src2d, dst2d, als[0], als[1], als[2], als[3],
        ald[0], ald[1], ald[2], ald[3])

    exself, ex_edges = pl.pallas_call(
        _tc4_body,
        out_shape=[jax.ShapeDtypeStruct((NP, HEADS), f32),
                   jax.ShapeDtypeStruct((HEADS, ERP, 128), f32)],
    )(tmax, eself, e_edges)

    outg, s0, s1, s2, s3 = _sc_gat_main(src2d, dst2d, ex_edges, h2h,
                                        zeros64, zeros1)
    outs = jnp.stack([s0, s1, s2, s3], axis=1)

    zs = pl.pallas_call(
        _tc5_body,
        grid=(NP // _TC5_B,),
        in_specs=[
            pl.BlockSpec((HEADS, _TC5_B, HID), lambda i: (0, i, 0)),
            pl.BlockSpec((_TC5_B, HEADS), lambda i: (i, 0)),
            pl.BlockSpec((_TC5_B, HEADS), lambda i: (i, 0)),
            pl.BlockSpec((HEADS, _TC5_B, HID), lambda i: (0, i, 0)),
            pl.BlockSpec((_TC5_B, 1), lambda i: (i, 0)),
            pl.BlockSpec((HID,), lambda i: (0,)),
            pl.BlockSpec((HID, 1), lambda i: (0, 0)),
        ],
        out_specs=pl.BlockSpec((_TC5_B, 1), lambda i: (i, 0)),
        out_shape=jax.ShapeDtypeStruct((NP, 1), f32),
    )(outg, outs, exself, h2h, dinv[:, None], ba, W3)
    zs = zs[:, 0]

    az0, az1 = _sc_gcn2(src2d, dst2d, zs, zeros1)

    out = pl.pallas_call(
        _tc6_body,
        out_shape=jax.ShapeDtypeStruct((NP, 1), f32),
    )(az0, az1, zs, dinv, b3)

    return out[:N]


def kernel(x, edge_index, W1, b1, Wa, a_src, a_dst, ba, W3, b3):
    return _run(x, edge_index, W1, b1, Wa, a_src, a_dst, ba, W3, b3)
